# baseline (speedup 1.0000x reference)
import functools
import jax
import jax.numpy as jnp
from jax import lax
from jax.experimental import pallas as pl
from jax.experimental.pallas import tpu as pltpu
from jax.experimental.pallas import tpu_sc as plsc

F32 = jnp.float32
NC, NS = 2, 16
NW = NC * NS
EK = 128
MB = 512
HIGH = lax.Precision.HIGHEST

_vmesh_cache = []


def _vmesh():
    if not _vmesh_cache:
        _vmesh_cache.append(
            plsc.VectorSubcoreMesh(core_axis_name="c", subcore_axis_name="s"))
    return _vmesh_cache[0]


def _rup(v, m):
    return ((v + m - 1) // m) * m


def _npad(n):
    return _rup(n + 1, 128)


def _sc_degree(dst, n):
    npad = _npad(n)
    ep = dst.shape[0]
    chunks = ep // (NW * EK)
    rp = npad // NS
    ones = jnp.ones((EK, 16), F32)
    zrows = jnp.zeros((npad, 16), F32)

    @functools.partial(
        pl.kernel,
        out_type=jax.ShapeDtypeStruct((NC, npad, 16), F32),
        mesh=_vmesh(),
        compiler_params=pltpu.CompilerParams(use_tc_tiling_on_sc=False),
        scratch_types=[
            pltpu.VMEM((EK,), jnp.int32),
            pltpu.VMEM((EK, 16), F32),
            pltpu.VMEM_SHARED((npad, 16), F32),
            pltpu.SemaphoreType.DMA,
        ],
    )
    def k(dst_hbm, ones_hbm, z_hbm, out_hbm, dstv, onesv, acc, sem):
        cid = lax.axis_index("c")
        sid = lax.axis_index("s")
        wid = cid * NS + sid
        pltpu.sync_copy(ones_hbm, onesv)
        pltpu.sync_copy(z_hbm.at[pl.ds(sid * rp, rp)], acc.at[pl.ds(sid * rp, rp)])
        plsc.subcore_barrier()
        ebase = wid * (ep // NW)

        @pl.loop(0, chunks)
        def _(j):
            pltpu.sync_copy(dst_hbm.at[pl.ds(ebase + j * EK, EK)], dstv)
            pltpu.sync_copy(onesv, acc.at[dstv], add=True)

        plsc.subcore_barrier()
        pltpu.sync_copy(acc.at[pl.ds(sid * rp, rp)],
                        out_hbm.at[cid].at[pl.ds(sid * rp, rp)])

    return k(dst, ones, zrows)


def _pipe(chunks, nb, start_g, start_c, wait_g, wait_c):
    if chunks <= nb:
        for jj in range(chunks):
            start_g(jj, jj)
        for jj in range(chunks):
            wait_g(jj)
            start_c(jj, jj)
            wait_c(jj)
        return
    lag = min(2, nb - 1)
    for r in range(nb):
        start_g(r, r)
    full = (chunks // nb) * nb

    @pl.loop(0, full, step=nb)
    def _(j):
        for r in range(nb):
            jj = j + r
            wait_g(r)
            start_c(jj, r)
            prr = (r - lag) % nb
            pp = jj - lag

            @pl.when(jnp.logical_and(pp >= 0, pp + nb < chunks))
            def _():
                wait_c(prr)
                start_g(pp + nb, prr)

    for jj in range(full, chunks):
        r = jj % nb
        wait_g(r)
        start_c(jj, r)
    for r in range(nb):
        wait_c(r)


def _sc_conv_scatter(hp, src, dst, n, cb, cc):
    npad = _npad(n)
    ep = src.shape[0]
    chunks = ep // (NW * EK)
    rp = npad // NS
    zrows = jnp.zeros((npad, cc), F32)
    src2 = src.reshape(-1, EK)
    dst2 = dst.reshape(-1, EK)
    nb = 2

    @functools.partial(
        pl.kernel,
        out_type=jax.ShapeDtypeStruct((NC, cb, npad, cc), F32),
        mesh=_vmesh(),
        compiler_params=pltpu.CompilerParams(use_tc_tiling_on_sc=False),
        scratch_types=(
            [pltpu.VMEM((chunks, EK), jnp.int32)] * 3
            + [pltpu.VMEM((EK, cc), F32)] * nb
            + [pltpu.VMEM_SHARED((npad, cc), F32)]
            + [pltpu.SemaphoreType.DMA] * (2 * nb)
        ),
    )
    def k(hp_hbm, src_hbm, dst_hbm, z_hbm, out_hbm, src2d, off2d, dst2d,
          *bufs_acc_sems):
        rows = bufs_acc_sems[:nb]
        acc = bufs_acc_sems[nb]
        sg = bufs_acc_sems[nb + 1:nb + 1 + nb]
        sc_ = bufs_acc_sems[nb + 1 + nb:]
        cid = lax.axis_index("c")
        sid = lax.axis_index("s")
        wid = cid * NS + sid
        cbase = wid * chunks
        pltpu.sync_copy(src_hbm.at[pl.ds(cbase, chunks)], src2d)
        pltpu.sync_copy(dst_hbm.at[pl.ds(cbase, chunks)], dst2d)

        @pl.loop(0, cb)
        def _(b):
            @pl.loop(0, chunks)
            def _(j):
                @pl.loop(0, EK, step=16)
                def _(t):
                    off2d[j, pl.ds(t, 16)] = src2d[j, pl.ds(t, 16)] + b * n

            pltpu.sync_copy(z_hbm.at[pl.ds(sid * rp, rp)],
                            acc.at[pl.ds(sid * rp, rp)])
            plsc.subcore_barrier()

            _pipe(
                chunks, nb,
                lambda jj, r: pltpu.async_copy(
                    hp_hbm.at[off2d.at[jj]], rows[r], sg[r]),
                lambda jj, r: pltpu.async_copy(
                    rows[r], acc.at[dst2d.at[jj]], sc_[r], add=True),
                lambda r: pltpu.make_async_copy(
                    hp_hbm.at[pl.ds(0, EK)], rows[r], sg[r]).wait(),
                lambda r: pltpu.make_async_copy(
                    rows[r], acc.at[dst2d.at[0]], sc_[r]).wait(),
            )

            plsc.subcore_barrier()
            pltpu.sync_copy(acc.at[pl.ds(sid * rp, rp)],
                            out_hbm.at[cid].at[b].at[pl.ds(sid * rp, rp)])
            plsc.subcore_barrier()

    return k(hp, src2, dst2, zrows)


def _sc_gather_wide(table, idx, cc):
    if cc <= 256:
        return _sc_gather(table, idx, cc)
    s = cc // 256
    mp = idx.shape[0]
    idxe = (idx[:, None] * s + jnp.arange(s, dtype=jnp.int32)[None, :]).reshape(-1)
    out = _sc_gather(table.reshape(table.shape[0] * s, 256), idxe, 256)
    return out.reshape(mp, cc)


def _sc_gather(table, idx, cc):
    mp = idx.shape[0]
    per_w = mp // NW
    chunks = per_w // EK
    nb = 4 if cc <= 128 else 3

    @functools.partial(
        pl.kernel,
        out_type=jax.ShapeDtypeStruct((mp, cc), F32),
        mesh=_vmesh(),
        compiler_params=pltpu.CompilerParams(use_tc_tiling_on_sc=False),
        scratch_types=(
            [pltpu.VMEM((chunks, EK), jnp.int32)]
            + [pltpu.VMEM((EK, cc), F32)] * nb
            + [pltpu.SemaphoreType.DMA] * (2 * nb)
        ),
    )
    def k(t_hbm, i_hbm, o_hbm, idx2d, *bufs_sems):
        rows = bufs_sems[:nb]
        sg = bufs_sems[nb:2 * nb]
        so = bufs_sems[2 * nb:]
        cid = lax.axis_index("c")
        sid = lax.axis_index("s")
        wid = cid * NS + sid
        base = wid * per_w
        cbase = wid * chunks
        pltpu.sync_copy(i_hbm.at[pl.ds(cbase, chunks)], idx2d)

        _pipe(
            chunks, nb,
            lambda jj, r: pltpu.async_copy(
                t_hbm.at[idx2d.at[jj]], rows[r], sg[r]),
            lambda jj, r: pltpu.async_copy(
                rows[r], o_hbm.at[pl.ds(base + jj * EK, EK)], so[r]),
            lambda r: pltpu.make_async_copy(
                t_hbm.at[pl.ds(0, EK)], rows[r], sg[r]).wait(),
            lambda r: pltpu.make_async_copy(
                rows[r], o_hbm.at[pl.ds(base, EK)], so[r]).wait(),
        )

    return k(table, idx.reshape(-1, EK))


def _tc_dis(cnt, n):
    def body(c_ref, o_ref):
        c = c_ref[0] + c_ref[1]
        o_ref[...] = lax.rsqrt(1.0 + c)

    grid = (pl.cdiv(n, MB),)
    return pl.pallas_call(
        body,
        grid=grid,
        in_specs=[pl.BlockSpec((NC, MB, 16), lambda m: (0, m, 0))],
        out_specs=pl.BlockSpec((MB, 16), lambda m: (m, 0)),
        out_shape=jax.ShapeDtypeStruct((n, 16), F32),
    )(cnt)


def _tc_mm(a, w, dis, cb):
    m, kk = a.shape
    c = w.shape[1]
    cc = c // cb

    def body(a_ref, w_ref, d_ref, h_ref, hp_ref):
        h = jnp.dot(a_ref[...], w_ref[...], preferred_element_type=F32,
                    precision=HIGH)
        h_ref[...] = h
        hp_ref[0] = d_ref[:, :1] * h

    grid = (pl.cdiv(m, MB), cb)
    return pl.pallas_call(
        body,
        grid=grid,
        in_specs=[
            pl.BlockSpec((MB, kk), lambda i, b: (i, 0)),
            pl.BlockSpec((kk, cc), lambda i, b: (0, b)),
            pl.BlockSpec((MB, 16), lambda i, b: (i, 0)),
        ],
        out_specs=[
            pl.BlockSpec((MB, cc), lambda i, b: (i, b)),
            pl.BlockSpec((1, MB, cc), lambda i, b: (b, i, 0)),
        ],
        out_shape=[
            jax.ShapeDtypeStruct((m, c), F32),
            jax.ShapeDtypeStruct((cb, m, cc), F32),
        ],
    )(a, w, dis)


def _tc_mm_fp2(a, w_rest, w_pool, pooled, dis, cb):
    m, kk = a.shape
    c = w_rest.shape[1]
    cc = c // cb

    def body(a_ref, wr_ref, wp_ref, p_ref, d_ref, h_ref, hp_ref):
        prow = jnp.dot(p_ref[:1], wp_ref[...], preferred_element_type=F32,
                       precision=HIGH)
        h = jnp.dot(a_ref[...], wr_ref[...], preferred_element_type=F32,
                    precision=HIGH) + prow
        h_ref[...] = h
        hp_ref[0] = d_ref[:, :1] * h

    grid = (pl.cdiv(m, MB), cb)
    gtd = w_pool.shape[0]
    return pl.pallas_call(
        body,
        grid=grid,
        in_specs=[
            pl.BlockSpec((MB, kk), lambda i, b: (i, 0)),
            pl.BlockSpec((kk, cc), lambda i, b: (0, b)),
            pl.BlockSpec((gtd, cc), lambda i, b: (0, b)),
            pl.BlockSpec((8, gtd), lambda i, b: (0, 0)),
            pl.BlockSpec((MB, 16), lambda i, b: (i, 0)),
        ],
        out_specs=[
            pl.BlockSpec((MB, cc), lambda i, b: (i, b)),
            pl.BlockSpec((1, MB, cc), lambda i, b: (b, i, 0)),
        ],
        out_shape=[
            jax.ShapeDtypeStruct((m, c), F32),
            jax.ShapeDtypeStruct((cb, m, cc), F32),
        ],
    )(a, w_rest, w_pool, pooled, dis)


def _tc_post(seg, h, dis, bias, cb):
    m, c = h.shape
    cc = c // cb

    def body(s_ref, h_ref, d_ref, b_ref, o_ref):
        s = s_ref[0, 0] + s_ref[1, 0]
        d = d_ref[:, :1]
        o_ref[...] = jax.nn.relu(d * s + d * d * h_ref[...] + b_ref[...])

    grid = (pl.cdiv(m, MB), cb)
    return pl.pallas_call(
        body,
        grid=grid,
        in_specs=[
            pl.BlockSpec((NC, 1, MB, cc), lambda i, b: (0, b, i, 0)),
            pl.BlockSpec((MB, cc), lambda i, b: (i, b)),
            pl.BlockSpec((MB, 16), lambda i, b: (i, 0)),
            pl.BlockSpec((1, cc), lambda i, b: (0, b)),
        ],
        out_specs=pl.BlockSpec((MB, cc), lambda i, b: (i, b)),
        out_shape=jax.ShapeDtypeStruct((m, c), F32),
    )(seg, h, dis, bias)


def _tc_knn3(pos_y, pos_x):
    ny = pos_y.shape[0]
    nx = pos_x.shape[0]

    def body(py_ref, px_ref, i_ref, w_ref):
        py = py_ref[...]
        px = px_ref[...]
        py2 = jnp.sum(py * py, axis=1, keepdims=True)
        px2 = lax.dot_general(jnp.ones((1, 3), F32), px * px,
                              (((1,), (1,)), ((), ())),
                              preferred_element_type=F32, precision=HIGH)
        cross = lax.dot_general(py, px, (((1,), (1,)), ((), ())),
                                preferred_element_type=F32, precision=HIGH)
        d = py2 - 2.0 * cross + px2
        col = lax.broadcasted_iota(jnp.int32, (MB, nx), 1)
        idxs = []
        ws = []
        for _ in range(3):
            mv = jnp.min(d, axis=1, keepdims=True)
            am = jnp.min(jnp.where(d == mv, col, nx), axis=1, keepdims=True)
            idxs.append(am)
            ws.append(1.0 / jnp.maximum(mv, 1e-16))
            d = jnp.where(col == am, 1e30, d)
        i_ref[...] = jnp.concatenate(idxs, axis=1)
        wst = jnp.concatenate(ws, axis=1)
        w_ref[...] = wst / jnp.sum(wst, axis=1, keepdims=True)

    grid = (pl.cdiv(ny, MB),)
    return pl.pallas_call(
        body,
        grid=grid,
        in_specs=[
            pl.BlockSpec((MB, 3), lambda i: (i, 0)),
            pl.BlockSpec((nx, 3), lambda i: (0, 0)),
        ],
        out_specs=[
            pl.BlockSpec((MB, 3), lambda i: (i, 0)),
            pl.BlockSpec((MB, 3), lambda i: (i, 0)),
        ],
        out_shape=[
            jax.ShapeDtypeStruct((ny, 3), jnp.int32),
            jax.ShapeDtypeStruct((ny, 3), F32),
        ],
    )(pos_y, pos_x)


def _tc_wsum(feats, wn):
    _, ny, cc = feats.shape

    def body(f_ref, w_ref, o_ref):
        o_ref[...] = (f_ref[0] * w_ref[:, 0:1] + f_ref[1] * w_ref[:, 1:2]
                      + f_ref[2] * w_ref[:, 2:3])

    grid = (pl.cdiv(ny, MB),)
    return pl.pallas_call(
        body,
        grid=grid,
        in_specs=[
            pl.BlockSpec((3, MB, cc), lambda i: (0, i, 0)),
            pl.BlockSpec((MB, 3), lambda i: (i, 0)),
        ],
        out_specs=pl.BlockSpec((MB, cc), lambda i: (i, 0)),
        out_shape=jax.ShapeDtypeStruct((ny, cc), F32),
    )(feats, wn)


def _tc_mlp_gp(x2pos, lins, bns):
    (w1, b1), (w2, b2), (w3, b3) = lins
    (g1, e1), (g2, e2) = bns
    gtd = w3.shape[1]

    def body(x_ref, w1_ref, b1_ref, g1_ref, e1_ref, w2_ref, b2_ref, g2_ref,
             e2_ref, w3_ref, b3_ref, o_ref):
        def bn_relu(h, g_ref, e_ref):
            mu = jnp.mean(h, axis=0, keepdims=True)
            var = jnp.mean((h - mu) ** 2, axis=0, keepdims=True)
            h = (h - mu) / jnp.sqrt(var + 1e-5) * g_ref[...] + e_ref[...]
            return jax.nn.relu(h)

        h = jnp.dot(x_ref[...], w1_ref[...], preferred_element_type=F32,
                    precision=HIGH) + b1_ref[...]
        h = bn_relu(h, g1_ref, e1_ref)
        h = jnp.dot(h, w2_ref[...], preferred_element_type=F32,
                    precision=HIGH) + b2_ref[...]
        h = bn_relu(h, g2_ref, e2_ref)
        h = jnp.dot(h, w3_ref[...], preferred_element_type=F32,
                    precision=HIGH) + b3_ref[...]
        pooled = jnp.max(h, axis=0, keepdims=True)
        o_ref[...] = jnp.broadcast_to(pooled, (8, gtd))

    args = (x2pos, w1, b1.reshape(1, -1), g1.reshape(1, -1), e1.reshape(1, -1),
            w2, b2.reshape(1, -1), g2.reshape(1, -1), e2.reshape(1, -1),
            w3, b3.reshape(1, -1))
    return pl.pallas_call(
        body,
        out_shape=jax.ShapeDtypeStruct((8, gtd), F32),
    )(*args)


def _tc_head(h, w1, b1, w2, b2):
    m = h.shape[0]
    co = w2.shape[1]

    def body(h_ref, w1_ref, b1_ref, w2_ref, b2_ref, o_ref):
        t = jax.nn.relu(jnp.dot(h_ref[...], w1_ref[...],
                                preferred_element_type=F32, precision=HIGH)
                        + b1_ref[...])
        o_ref[...] = jnp.dot(t, w2_ref[...], preferred_element_type=F32,
                             precision=HIGH) + b2_ref[...]

    grid = (pl.cdiv(m, MB),)
    return pl.pallas_call(
        body,
        grid=grid,
        in_specs=[
            pl.BlockSpec((MB, w1.shape[0]), lambda i: (i, 0)),
            pl.BlockSpec(w1.shape, lambda i: (0, 0)),
            pl.BlockSpec((1, w1.shape[1]), lambda i: (0, 0)),
            pl.BlockSpec(w2.shape, lambda i: (0, 0)),
            pl.BlockSpec((1, co), lambda i: (0, 0)),
        ],
        out_specs=pl.BlockSpec((MB, co), lambda i: (i, 0)),
        out_shape=jax.ShapeDtypeStruct((m, co), F32),
    )(h, w1, b1.reshape(1, -1), w2, b2.reshape(1, -1))


def _pad_edges(ei, n):
    e = ei.shape[1]
    ep = _rup(e, NW * EK)
    src = jnp.concatenate([ei[0].astype(jnp.int32),
                           jnp.zeros((ep - e,), jnp.int32)])
    dst = jnp.concatenate([ei[1].astype(jnp.int32),
                           jnp.full((ep - e,), n, jnp.int32)])
    return src, dst


def _pad_idx(idx):
    mpad = _rup(idx.shape[0], NW * EK)
    return jnp.concatenate([idx.astype(jnp.int32),
                            jnp.zeros((mpad - idx.shape[0],), jnp.int32)])


def _pad_cols(a, cc):
    if a.shape[1] == cc:
        return a
    return jnp.concatenate(
        [a, jnp.zeros((a.shape[0], cc - a.shape[1]), F32)], axis=1)


def _conv_block(a, convs, src, dst, dis, n, cbs):
    h_act = a
    for (w, b), cb in zip(convs, cbs):
        c = w.shape[1]
        cc = c // cb
        h, hp = _tc_mm(h_act, w, dis, cb)
        seg = _sc_conv_scatter(hp.reshape(cb * n, cc), src, dst, n, cb, cc)
        h_act = _tc_post(seg, h, dis, b.reshape(1, -1), cb)
    return h_act


def kernel(x, pos, params, batch, idx0, idx1, edge_index0, edge_index1,
           edge_index2):
    n0 = x.shape[0]
    n1 = idx0.shape[0]
    n2 = idx1.shape[0]
    in_c = x.shape[1]

    src0, dst0 = _pad_edges(edge_index0, n0)
    src1, dst1 = _pad_edges(edge_index1, n1)
    src2, dst2 = _pad_edges(edge_index2, n2)

    dis0 = _tc_dis(_sc_degree(dst0, n0), n0)
    dis1 = _tc_dis(_sc_degree(dst1, n1), n1)
    dis2 = _tc_dis(_sc_degree(dst2, n2), n2)

    h = _conv_block(x, params['sa1'], src0, dst0, dis0, n0, cbs=[1, 1, 1])

    tbl = _pad_cols(jnp.concatenate([h, pos], axis=1), 80)
    g1 = _sc_gather(tbl, _pad_idx(idx0), 80)[:n1]
    x1 = g1[:, :h.shape[1]]
    pos1 = g1[:, h.shape[1]:h.shape[1] + 3]
    a1 = g1[:, :h.shape[1] + 3]

    h = _conv_block(a1, params['sa2'], src1, dst1, dis1, n1, cbs=[1, 1, 1])

    tbl = _pad_cols(jnp.concatenate([h, pos1], axis=1), 144)
    g2 = _sc_gather(tbl, _pad_idx(idx1), 144)[:n2]
    x2pos = g2[:, :h.shape[1] + 3]
    pos2 = g2[:, h.shape[1]:h.shape[1] + 3]

    xb = _conv_block(x2pos, params['bn'], src2, dst2, dis2, n2, cbs=[1, 1, 1])

    pooled = _tc_mlp_gp(x2pos, params['gp_lin'], params['gp_bn'])

    idxk, wn = _tc_knn3(pos1, pos2)
    flat = _pad_idx(jnp.transpose(idxk).reshape(-1))
    feats = _sc_gather_wide(xb, flat, xb.shape[1])[:3 * n1].reshape(3, n1, -1)
    interp_xb = _tc_wsum(feats, wn)
    a_small = jnp.concatenate([interp_xb, x1], axis=1)

    (w, b) = params['fp2'][0]
    gtd = pooled.shape[1]
    h, hp = _tc_mm_fp2(a_small, w[gtd:], w[:gtd], pooled, dis1, cb=4)
    cc = w.shape[1] // 4
    seg = _sc_conv_scatter(hp.reshape(4 * n1, cc), src1, dst1, n1, 4, cc)
    h = _tc_post(seg, h, dis1, b.reshape(1, -1), 4)
    h = _conv_block(h, params['fp2'][1:], src1, dst1, dis1, n1, cbs=[4, 2])

    idxk, wn = _tc_knn3(pos, pos1)
    flat = _pad_idx(jnp.transpose(idxk).reshape(-1))
    feats = _sc_gather_wide(h, flat, h.shape[1])[:3 * n0].reshape(3, n0, -1)
    interp = _tc_wsum(feats, wn)
    a0 = jnp.concatenate([interp, x[:, :in_c]], axis=1)

    h = _conv_block(a0, params['fp1'], src0, dst0, dis0, n0, cbs=[2, 2, 1])

    (w1, b1), (w2, b2) = params['head']
    return _tc_head(h, w1, b1, w2, b2)


# speedup vs baseline: 1.0327x; 1.0327x over previous
import functools
import jax
import jax.numpy as jnp
from jax import lax
from jax.experimental import pallas as pl
from jax.experimental.pallas import tpu as pltpu
from jax.experimental.pallas import tpu_sc as plsc

F32 = jnp.float32
NC, NS = 2, 16
NW = NC * NS
EK = 128
MB = 512
HIGH = lax.Precision.HIGHEST

_vmesh_cache = []


def _vmesh():
    if not _vmesh_cache:
        _vmesh_cache.append(
            plsc.VectorSubcoreMesh(core_axis_name="c", subcore_axis_name="s"))
    return _vmesh_cache[0]


def _rup(v, m):
    return ((v + m - 1) // m) * m


def _npad(n):
    return _rup(n + 1, 128)


def _sc_degree(dst, n):
    npad = _npad(n)
    ep = dst.shape[0]
    chunks = ep // (NW * EK)
    rp = npad // NS
    ones = jnp.ones((EK, 16), F32)
    zrows = jnp.zeros((npad, 16), F32)

    @functools.partial(
        pl.kernel,
        out_type=jax.ShapeDtypeStruct((NC, npad, 16), F32),
        mesh=_vmesh(),
        compiler_params=pltpu.CompilerParams(use_tc_tiling_on_sc=False),
        scratch_types=[
            pltpu.VMEM((chunks, EK), jnp.int32),
            pltpu.VMEM((EK, 16), F32),
            pltpu.VMEM_SHARED((npad, 16), F32),
            pltpu.SemaphoreType.DMA,
        ],
    )
    def k(dst_hbm, ones_hbm, z_hbm, out_hbm, dst2d, onesv, acc, sem):
        cid = lax.axis_index("c")
        sid = lax.axis_index("s")
        wid = cid * NS + sid
        pltpu.sync_copy(dst_hbm.at[pl.ds(wid * chunks, chunks)], dst2d)
        pltpu.sync_copy(ones_hbm, onesv)
        pltpu.sync_copy(z_hbm.at[pl.ds(sid * rp, rp)], acc.at[pl.ds(sid * rp, rp)])
        plsc.subcore_barrier()

        for j0 in range(0, chunks, 8):
            jw = min(8, chunks - j0)
            for j in range(j0, j0 + jw):
                pltpu.async_copy(onesv, acc.at[dst2d.at[j]], sem, add=True)
            for _ in range(jw):
                pltpu.make_async_copy(onesv, acc.at[dst2d.at[0]], sem).wait()

        plsc.subcore_barrier()
        pltpu.sync_copy(acc.at[pl.ds(sid * rp, rp)],
                        out_hbm.at[cid].at[pl.ds(sid * rp, rp)])

    return k(dst.reshape(-1, EK), ones, zrows)


def _sc_conv_scatter(hp, src, dst, n, cb, cc):
    npad = _npad(n)
    ep = src.shape[0]
    chunks = ep // (NW * EK)
    rp = npad // NS
    zrows = jnp.zeros((npad, cc), F32)
    src2 = src.reshape(-1, EK)
    dst2 = dst.reshape(-1, EK)
    c2 = chunks - (chunks % 2)

    @functools.partial(
        pl.kernel,
        out_type=jax.ShapeDtypeStruct((NC, cb, npad, cc), F32),
        mesh=_vmesh(),
        compiler_params=pltpu.CompilerParams(use_tc_tiling_on_sc=False),
        scratch_types=[
            pltpu.VMEM((chunks, EK), jnp.int32),
            pltpu.VMEM((chunks, EK), jnp.int32),
            pltpu.VMEM((chunks, EK), jnp.int32),
            pltpu.VMEM((EK, cc), F32),
            pltpu.VMEM((EK, cc), F32),
            pltpu.VMEM_SHARED((npad, cc), F32),
            pltpu.SemaphoreType.DMA,
            pltpu.SemaphoreType.DMA,
        ],
    )
    def k(hp_hbm, src_hbm, dst_hbm, z_hbm, out_hbm, src2d, off2d, dst2d,
          rowa, rowb, acc, sema, semb):
        cid = lax.axis_index("c")
        sid = lax.axis_index("s")
        wid = cid * NS + sid
        cbase = wid * chunks
        pltpu.sync_copy(src_hbm.at[pl.ds(cbase, chunks)], src2d)
        pltpu.sync_copy(dst_hbm.at[pl.ds(cbase, chunks)], dst2d)

        def wait_g(buf, sem):
            pltpu.make_async_copy(hp_hbm.at[pl.ds(0, EK)], buf, sem).wait()

        def scat(jj, buf):
            pltpu.sync_copy(buf, acc.at[dst2d.at[jj]], add=True)

        for b in range(cb):
            idx2 = src2d if b == 0 else off2d
            if b > 0:
                @pl.loop(0, chunks)
                def _(j):
                    @pl.loop(0, EK, step=16)
                    def _(t):
                        off2d[j, pl.ds(t, 16)] = src2d[j, pl.ds(t, 16)] + b * n

            pltpu.sync_copy(z_hbm.at[pl.ds(sid * rp, rp)],
                            acc.at[pl.ds(sid * rp, rp)])
            plsc.subcore_barrier()

            def start_g(jj, buf, sem, idx2=idx2):
                pltpu.async_copy(hp_hbm.at[idx2.at[jj]], buf, sem)

            if c2 > 0:
                start_g(0, rowa, sema)

                @pl.loop(0, c2, step=2)
                def _(j):
                    start_g(j + 1, rowb, semb)
                    wait_g(rowa, sema)
                    scat(j, rowa)

                    @pl.when(j + 2 < c2)
                    def _():
                        start_g(j + 2, rowa, sema)

                    wait_g(rowb, semb)
                    scat(j + 1, rowb)

            if chunks % 2:
                start_g(chunks - 1, rowa, sema)
                wait_g(rowa, sema)
                scat(chunks - 1, rowa)

            plsc.subcore_barrier()
            pltpu.sync_copy(acc.at[pl.ds(sid * rp, rp)],
                            out_hbm.at[cid].at[b].at[pl.ds(sid * rp, rp)])
            plsc.subcore_barrier()

    return k(hp, src2, dst2, zrows)


def _sc_gather_wide(table, idx, cc):
    if cc <= 256:
        return _sc_gather(table, idx, cc)
    s = cc // 256
    mp = idx.shape[0]
    idxe = (idx[:, None] * s + jnp.arange(s, dtype=jnp.int32)[None, :]).reshape(-1)
    out = _sc_gather(table.reshape(table.shape[0] * s, 256), idxe, 256)
    return out.reshape(mp, cc)


def _sc_gather(table, idx, cc):
    mp = idx.shape[0]
    per_w = mp // NW
    chunks = per_w // EK

    @functools.partial(
        pl.kernel,
        out_type=jax.ShapeDtypeStruct((mp, cc), F32),
        mesh=_vmesh(),
        compiler_params=pltpu.CompilerParams(use_tc_tiling_on_sc=False),
        scratch_types=[
            pltpu.VMEM((chunks, EK), jnp.int32),
            pltpu.VMEM((EK, cc), F32),
            pltpu.VMEM((EK, cc), F32),
            pltpu.SemaphoreType.DMA,
            pltpu.SemaphoreType.DMA,
        ],
    )
    def k(t_hbm, i_hbm, o_hbm, idx2d, rowa, rowb, sema, semb):
        cid = lax.axis_index("c")
        sid = lax.axis_index("s")
        wid = cid * NS + sid
        base = wid * per_w
        cbase = wid * chunks
        pltpu.sync_copy(i_hbm.at[pl.ds(cbase, chunks)], idx2d)

        def start_g(jj, buf, sem):
            pltpu.async_copy(t_hbm.at[idx2d.at[jj]], buf, sem)

        def wait_g(buf, sem):
            pltpu.make_async_copy(t_hbm.at[pl.ds(0, EK)], buf, sem).wait()

        def store(jj, buf):
            pltpu.sync_copy(buf, o_hbm.at[pl.ds(base + jj * EK, EK)])

        c2 = chunks - (chunks % 2)
        if c2 > 0:
            start_g(0, rowa, sema)

            @pl.loop(0, c2, step=2)
            def _(j):
                start_g(j + 1, rowb, semb)
                wait_g(rowa, sema)
                store(j, rowa)

                @pl.when(j + 2 < c2)
                def _():
                    start_g(j + 2, rowa, sema)

                wait_g(rowb, semb)
                store(j + 1, rowb)

        if chunks % 2:
            start_g(chunks - 1, rowa, sema)
            wait_g(rowa, sema)
            store(chunks - 1, rowa)

    return k(table, idx.reshape(-1, EK))


def _tc_dis(cnt, n):
    def body(c_ref, o_ref):
        c = c_ref[0] + c_ref[1]
        o_ref[...] = lax.rsqrt(1.0 + c)

    grid = (pl.cdiv(n, MB),)
    return pl.pallas_call(
        body,
        grid=grid,
        in_specs=[pl.BlockSpec((NC, MB, 16), lambda m: (0, m, 0))],
        out_specs=pl.BlockSpec((MB, 16), lambda m: (m, 0)),
        out_shape=jax.ShapeDtypeStruct((n, 16), F32),
    )(cnt)


def _tc_mm(a, w, dis, cb):
    m, kk = a.shape
    c = w.shape[1]
    cc = c // cb

    def body(a_ref, w_ref, d_ref, h_ref, hp_ref):
        h = jnp.dot(a_ref[...], w_ref[...], preferred_element_type=F32,
                    precision=HIGH)
        h_ref[...] = h
        hp_ref[0] = d_ref[:, :1] * h

    grid = (pl.cdiv(m, MB), cb)
    return pl.pallas_call(
        body,
        grid=grid,
        in_specs=[
            pl.BlockSpec((MB, kk), lambda i, b: (i, 0)),
            pl.BlockSpec((kk, cc), lambda i, b: (0, b)),
            pl.BlockSpec((MB, 16), lambda i, b: (i, 0)),
        ],
        out_specs=[
            pl.BlockSpec((MB, cc), lambda i, b: (i, b)),
            pl.BlockSpec((1, MB, cc), lambda i, b: (b, i, 0)),
        ],
        out_shape=[
            jax.ShapeDtypeStruct((m, c), F32),
            jax.ShapeDtypeStruct((cb, m, cc), F32),
        ],
    )(a, w, dis)


def _tc_mm_fp2(a, w_rest, w_pool, pooled, dis, cb):
    m, kk = a.shape
    c = w_rest.shape[1]
    cc = c // cb

    def body(a_ref, wr_ref, wp_ref, p_ref, d_ref, h_ref, hp_ref):
        prow = jnp.dot(p_ref[:1], wp_ref[...], preferred_element_type=F32,
                       precision=HIGH)
        h = jnp.dot(a_ref[...], wr_ref[...], preferred_element_type=F32,
                    precision=HIGH) + prow
        h_ref[...] = h
        hp_ref[0] = d_ref[:, :1] * h

    grid = (pl.cdiv(m, MB), cb)
    gtd = w_pool.shape[0]
    return pl.pallas_call(
        body,
        grid=grid,
        in_specs=[
            pl.BlockSpec((MB, kk), lambda i, b: (i, 0)),
            pl.BlockSpec((kk, cc), lambda i, b: (0, b)),
            pl.BlockSpec((gtd, cc), lambda i, b: (0, b)),
            pl.BlockSpec((8, gtd), lambda i, b: (0, 0)),
            pl.BlockSpec((MB, 16), lambda i, b: (i, 0)),
        ],
        out_specs=[
            pl.BlockSpec((MB, cc), lambda i, b: (i, b)),
            pl.BlockSpec((1, MB, cc), lambda i, b: (b, i, 0)),
        ],
        out_shape=[
            jax.ShapeDtypeStruct((m, c), F32),
            jax.ShapeDtypeStruct((cb, m, cc), F32),
        ],
    )(a, w_rest, w_pool, pooled, dis)


def _tc_post(seg, h, dis, bias, cb):
    m, c = h.shape
    cc = c // cb

    def body(s_ref, h_ref, d_ref, b_ref, o_ref):
        s = s_ref[0, 0] + s_ref[1, 0]
        d = d_ref[:, :1]
        o_ref[...] = jax.nn.relu(d * s + d * d * h_ref[...] + b_ref[...])

    grid = (pl.cdiv(m, MB), cb)
    return pl.pallas_call(
        body,
        grid=grid,
        in_specs=[
            pl.BlockSpec((NC, 1, MB, cc), lambda i, b: (0, b, i, 0)),
            pl.BlockSpec((MB, cc), lambda i, b: (i, b)),
            pl.BlockSpec((MB, 16), lambda i, b: (i, 0)),
            pl.BlockSpec((1, cc), lambda i, b: (0, b)),
        ],
        out_specs=pl.BlockSpec((MB, cc), lambda i, b: (i, b)),
        out_shape=jax.ShapeDtypeStruct((m, c), F32),
    )(seg, h, dis, bias)


def _tc_knn3(pos_y, pos_x):
    ny = pos_y.shape[0]
    nx = pos_x.shape[0]

    def body(py_ref, px_ref, i_ref, w_ref):
        py = py_ref[...]
        px = px_ref[...]
        py2 = jnp.sum(py * py, axis=1, keepdims=True)
        px2 = lax.dot_general(jnp.ones((1, 3), F32), px * px,
                              (((1,), (1,)), ((), ())),
                              preferred_element_type=F32, precision=HIGH)
        cross = lax.dot_general(py, px, (((1,), (1,)), ((), ())),
                                preferred_element_type=F32, precision=HIGH)
        d = py2 - 2.0 * cross + px2
        col = lax.broadcasted_iota(jnp.int32, (MB, nx), 1)
        idxs = []
        ws = []
        for _ in range(3):
            mv = jnp.min(d, axis=1, keepdims=True)
            am = jnp.min(jnp.where(d == mv, col, nx), axis=1, keepdims=True)
            idxs.append(am)
            ws.append(1.0 / jnp.maximum(mv, 1e-16))
            d = jnp.where(col == am, 1e30, d)
        i_ref[...] = jnp.concatenate(idxs, axis=1)
        wst = jnp.concatenate(ws, axis=1)
        w_ref[...] = wst / jnp.sum(wst, axis=1, keepdims=True)

    grid = (pl.cdiv(ny, MB),)
    return pl.pallas_call(
        body,
        grid=grid,
        in_specs=[
            pl.BlockSpec((MB, 3), lambda i: (i, 0)),
            pl.BlockSpec((nx, 3), lambda i: (0, 0)),
        ],
        out_specs=[
            pl.BlockSpec((MB, 3), lambda i: (i, 0)),
            pl.BlockSpec((MB, 3), lambda i: (i, 0)),
        ],
        out_shape=[
            jax.ShapeDtypeStruct((ny, 3), jnp.int32),
            jax.ShapeDtypeStruct((ny, 3), F32),
        ],
    )(pos_y, pos_x)


def _tc_wsum(feats, wn):
    _, ny, cc = feats.shape

    def body(f_ref, w_ref, o_ref):
        o_ref[...] = (f_ref[0] * w_ref[:, 0:1] + f_ref[1] * w_ref[:, 1:2]
                      + f_ref[2] * w_ref[:, 2:3])

    grid = (pl.cdiv(ny, MB),)
    return pl.pallas_call(
        body,
        grid=grid,
        in_specs=[
            pl.BlockSpec((3, MB, cc), lambda i: (0, i, 0)),
            pl.BlockSpec((MB, 3), lambda i: (i, 0)),
        ],
        out_specs=pl.BlockSpec((MB, cc), lambda i: (i, 0)),
        out_shape=jax.ShapeDtypeStruct((ny, cc), F32),
    )(feats, wn)


def _tc_mlp_gp(x2pos, lins, bns):
    (w1, b1), (w2, b2), (w3, b3) = lins
    (g1, e1), (g2, e2) = bns
    gtd = w3.shape[1]

    def body(x_ref, w1_ref, b1_ref, g1_ref, e1_ref, w2_ref, b2_ref, g2_ref,
             e2_ref, w3_ref, b3_ref, o_ref):
        def bn_relu(h, g_ref, e_ref):
            mu = jnp.mean(h, axis=0, keepdims=True)
            var = jnp.mean((h - mu) ** 2, axis=0, keepdims=True)
            h = (h - mu) / jnp.sqrt(var + 1e-5) * g_ref[...] + e_ref[...]
            return jax.nn.relu(h)

        h = jnp.dot(x_ref[...], w1_ref[...], preferred_element_type=F32,
                    precision=HIGH) + b1_ref[...]
        h = bn_relu(h, g1_ref, e1_ref)
        h = jnp.dot(h, w2_ref[...], preferred_element_type=F32,
                    precision=HIGH) + b2_ref[...]
        h = bn_relu(h, g2_ref, e2_ref)
        h = jnp.dot(h, w3_ref[...], preferred_element_type=F32,
                    precision=HIGH) + b3_ref[...]
        pooled = jnp.max(h, axis=0, keepdims=True)
        o_ref[...] = jnp.broadcast_to(pooled, (8, gtd))

    args = (x2pos, w1, b1.reshape(1, -1), g1.reshape(1, -1), e1.reshape(1, -1),
            w2, b2.reshape(1, -1), g2.reshape(1, -1), e2.reshape(1, -1),
            w3, b3.reshape(1, -1))
    return pl.pallas_call(
        body,
        out_shape=jax.ShapeDtypeStruct((8, gtd), F32),
    )(*args)


def _tc_head(h, w1, b1, w2, b2):
    m = h.shape[0]
    co = w2.shape[1]

    def body(h_ref, w1_ref, b1_ref, w2_ref, b2_ref, o_ref):
        t = jax.nn.relu(jnp.dot(h_ref[...], w1_ref[...],
                                preferred_element_type=F32, precision=HIGH)
                        + b1_ref[...])
        o_ref[...] = jnp.dot(t, w2_ref[...], preferred_element_type=F32,
                             precision=HIGH) + b2_ref[...]

    grid = (pl.cdiv(m, MB),)
    return pl.pallas_call(
        body,
        grid=grid,
        in_specs=[
            pl.BlockSpec((MB, w1.shape[0]), lambda i: (i, 0)),
            pl.BlockSpec(w1.shape, lambda i: (0, 0)),
            pl.BlockSpec((1, w1.shape[1]), lambda i: (0, 0)),
            pl.BlockSpec(w2.shape, lambda i: (0, 0)),
            pl.BlockSpec((1, co), lambda i: (0, 0)),
        ],
        out_specs=pl.BlockSpec((MB, co), lambda i: (i, 0)),
        out_shape=jax.ShapeDtypeStruct((m, co), F32),
    )(h, w1, b1.reshape(1, -1), w2, b2.reshape(1, -1))


def _pad_edges(ei, n):
    e = ei.shape[1]
    ep = _rup(e, NW * EK)
    src = jnp.concatenate([ei[0].astype(jnp.int32),
                           jnp.zeros((ep - e,), jnp.int32)])
    dst = jnp.concatenate([ei[1].astype(jnp.int32),
                           jnp.full((ep - e,), n, jnp.int32)])
    return src, dst


def _pad_idx(idx):
    mpad = _rup(idx.shape[0], NW * EK)
    return jnp.concatenate([idx.astype(jnp.int32),
                            jnp.zeros((mpad - idx.shape[0],), jnp.int32)])


def _pad_cols(a, cc):
    if a.shape[1] == cc:
        return a
    return jnp.concatenate(
        [a, jnp.zeros((a.shape[0], cc - a.shape[1]), F32)], axis=1)


def _conv_block(a, convs, src, dst, dis, n, cbs):
    h_act = a
    for (w, b), cb in zip(convs, cbs):
        c = w.shape[1]
        cc = c // cb
        h, hp = _tc_mm(h_act, w, dis, cb)
        seg = _sc_conv_scatter(hp.reshape(cb * n, cc), src, dst, n, cb, cc)
        h_act = _tc_post(seg, h, dis, b.reshape(1, -1), cb)
    return h_act


def kernel(x, pos, params, batch, idx0, idx1, edge_index0, edge_index1,
           edge_index2):
    n0 = x.shape[0]
    n1 = idx0.shape[0]
    n2 = idx1.shape[0]
    in_c = x.shape[1]

    src0, dst0 = _pad_edges(edge_index0, n0)
    src1, dst1 = _pad_edges(edge_index1, n1)
    src2, dst2 = _pad_edges(edge_index2, n2)

    dis0 = _tc_dis(_sc_degree(dst0, n0), n0)
    dis1 = _tc_dis(_sc_degree(dst1, n1), n1)
    dis2 = _tc_dis(_sc_degree(dst2, n2), n2)

    h = _conv_block(x, params['sa1'], src0, dst0, dis0, n0, cbs=[1, 1, 1])

    tbl = _pad_cols(jnp.concatenate([h, pos], axis=1), 80)
    g1 = _sc_gather(tbl, _pad_idx(idx0), 80)[:n1]
    x1 = g1[:, :h.shape[1]]
    pos1 = g1[:, h.shape[1]:h.shape[1] + 3]
    a1 = g1[:, :h.shape[1] + 3]

    h = _conv_block(a1, params['sa2'], src1, dst1, dis1, n1, cbs=[1, 1, 1])

    tbl = _pad_cols(jnp.concatenate([h, pos1], axis=1), 144)
    g2 = _sc_gather(tbl, _pad_idx(idx1), 144)[:n2]
    x2pos = g2[:, :h.shape[1] + 3]
    pos2 = g2[:, h.shape[1]:h.shape[1] + 3]

    xb = _conv_block(x2pos, params['bn'], src2, dst2, dis2, n2, cbs=[1, 1, 1])

    pooled = _tc_mlp_gp(x2pos, params['gp_lin'], params['gp_bn'])

    idxk, wn = _tc_knn3(pos1, pos2)
    flat = _pad_idx(jnp.transpose(idxk).reshape(-1))
    feats = _sc_gather_wide(xb, flat, xb.shape[1])[:3 * n1].reshape(3, n1, -1)
    interp_xb = _tc_wsum(feats, wn)
    a_small = jnp.concatenate([interp_xb, x1], axis=1)

    (w, b) = params['fp2'][0]
    gtd = pooled.shape[1]
    h, hp = _tc_mm_fp2(a_small, w[gtd:], w[:gtd], pooled, dis1, cb=4)
    cc = w.shape[1] // 4
    seg = _sc_conv_scatter(hp.reshape(4 * n1, cc), src1, dst1, n1, 4, cc)
    h = _tc_post(seg, h, dis1, b.reshape(1, -1), 4)
    h = _conv_block(h, params['fp2'][1:], src1, dst1, dis1, n1, cbs=[4, 2])

    idxk, wn = _tc_knn3(pos, pos1)
    flat = _pad_idx(jnp.transpose(idxk).reshape(-1))
    feats = _sc_gather_wide(h, flat, h.shape[1])[:3 * n0].reshape(3, n0, -1)
    interp = _tc_wsum(feats, wn)
    a0 = jnp.concatenate([interp, x[:, :in_c]], axis=1)

    h = _conv_block(a0, params['fp1'], src0, dst0, dis0, n0, cbs=[2, 2, 1])

    (w1, b1), (w2, b2) = params['head']
    return _tc_head(h, w1, b1, w2, b2)


# speedup vs baseline: 1.0901x; 1.0556x over previous
import functools
import jax
import jax.numpy as jnp
from jax import lax
from jax.experimental import pallas as pl
from jax.experimental.pallas import tpu as pltpu
from jax.experimental.pallas import tpu_sc as plsc

F32 = jnp.float32
NC, NS = 2, 16
NW = NC * NS
EK = 128
MB = 512
HIGH = lax.Precision.HIGHEST

_vmesh_cache = []


def _vmesh():
    if not _vmesh_cache:
        _vmesh_cache.append(
            plsc.VectorSubcoreMesh(core_axis_name="c", subcore_axis_name="s"))
    return _vmesh_cache[0]


def _rup(v, m):
    return ((v + m - 1) // m) * m


def _npad(n):
    return _rup(n + 1, 128)


def _sc_degree(dst, n):
    npad = _npad(n)
    ep = dst.shape[0]
    chunks = ep // (NW * EK)
    rp = npad // NS
    ones = jnp.ones((EK, 16), F32)
    zrows = jnp.zeros((npad, 16), F32)

    @functools.partial(
        pl.kernel,
        out_type=jax.ShapeDtypeStruct((NC, npad, 16), F32),
        mesh=_vmesh(),
        compiler_params=pltpu.CompilerParams(use_tc_tiling_on_sc=False),
        scratch_types=[
            pltpu.VMEM((chunks, EK), jnp.int32),
            pltpu.VMEM((EK, 16), F32),
            pltpu.VMEM_SHARED((npad, 16), F32),
            pltpu.SemaphoreType.DMA,
        ],
    )
    def k(dst_hbm, ones_hbm, z_hbm, out_hbm, dst2d, onesv, acc, sem):
        cid = lax.axis_index("c")
        sid = lax.axis_index("s")
        wid = cid * NS + sid
        pltpu.sync_copy(dst_hbm.at[pl.ds(wid * chunks, chunks)], dst2d)
        pltpu.sync_copy(ones_hbm, onesv)
        pltpu.sync_copy(z_hbm.at[pl.ds(sid * rp, rp)], acc.at[pl.ds(sid * rp, rp)])
        plsc.subcore_barrier()

        for j0 in range(0, chunks, 8):
            jw = min(8, chunks - j0)
            for j in range(j0, j0 + jw):
                pltpu.async_copy(onesv, acc.at[dst2d.at[j]], sem, add=True)
            for _ in range(jw):
                pltpu.make_async_copy(onesv, acc.at[dst2d.at[0]], sem).wait()

        plsc.subcore_barrier()
        pltpu.sync_copy(acc.at[pl.ds(sid * rp, rp)],
                        out_hbm.at[cid].at[pl.ds(sid * rp, rp)])

    return k(dst.reshape(-1, EK), ones, zrows)


def _sc_conv_scatter(hp, src, dst, n, cb, cc):
    npad = _npad(n)
    ep = src.shape[0]
    chunks = ep // (NW * EK)
    rp = npad // NS
    zrows = jnp.zeros((npad, cc), F32)
    src2 = src.reshape(-1, EK)
    dst2 = dst.reshape(-1, EK)
    c2 = chunks - (chunks % 2)

    @functools.partial(
        pl.kernel,
        out_type=jax.ShapeDtypeStruct((NC, cb, npad, cc), F32),
        mesh=_vmesh(),
        compiler_params=pltpu.CompilerParams(use_tc_tiling_on_sc=False),
        scratch_types=[
            pltpu.VMEM((chunks, EK), jnp.int32),
            pltpu.VMEM((chunks, EK), jnp.int32),
            pltpu.VMEM((chunks, EK), jnp.int32),
            pltpu.VMEM((EK, cc), F32),
            pltpu.VMEM((EK, cc), F32),
            pltpu.VMEM_SHARED((npad, cc), F32),
            pltpu.SemaphoreType.DMA,
            pltpu.SemaphoreType.DMA,
        ],
    )
    def k(hp_hbm, src_hbm, dst_hbm, z_hbm, out_hbm, src2d, off2d, dst2d,
          rowa, rowb, acc, sema, semb):
        cid = lax.axis_index("c")
        sid = lax.axis_index("s")
        wid = cid * NS + sid
        cbase = wid * chunks
        pltpu.sync_copy(src_hbm.at[pl.ds(cbase, chunks)], src2d)
        pltpu.sync_copy(dst_hbm.at[pl.ds(cbase, chunks)], dst2d)

        def wait_g(buf, sem):
            pltpu.make_async_copy(hp_hbm.at[pl.ds(0, EK)], buf, sem).wait()

        def scat(jj, buf):
            pltpu.sync_copy(buf, acc.at[dst2d.at[jj]], add=True)

        for b in range(cb):
            idx2 = src2d if b == 0 else off2d
            if b > 0:
                @pl.loop(0, chunks)
                def _(j):
                    @pl.loop(0, EK, step=16)
                    def _(t):
                        off2d[j, pl.ds(t, 16)] = src2d[j, pl.ds(t, 16)] + b * n

            pltpu.sync_copy(z_hbm.at[pl.ds(sid * rp, rp)],
                            acc.at[pl.ds(sid * rp, rp)])
            plsc.subcore_barrier()

            def start_g(jj, buf, sem, idx2=idx2):
                pltpu.async_copy(hp_hbm.at[idx2.at[jj]], buf, sem)

            if c2 > 0:
                start_g(0, rowa, sema)

                @pl.loop(0, c2, step=2)
                def _(j):
                    start_g(j + 1, rowb, semb)
                    wait_g(rowa, sema)
                    scat(j, rowa)

                    @pl.when(j + 2 < c2)
                    def _():
                        start_g(j + 2, rowa, sema)

                    wait_g(rowb, semb)
                    scat(j + 1, rowb)

            if chunks % 2:
                start_g(chunks - 1, rowa, sema)
                wait_g(rowa, sema)
                scat(chunks - 1, rowa)

            plsc.subcore_barrier()
            pltpu.sync_copy(acc.at[pl.ds(sid * rp, rp)],
                            out_hbm.at[cid].at[b].at[pl.ds(sid * rp, rp)])
            plsc.subcore_barrier()

    return k(hp, src2, dst2, zrows)


def _sc_conv_scatter_split(hp, src, dst, n, cb, cc):
    npad = _npad(n)
    ep = src.shape[0]
    chunks = ep // (NS * EK)
    rp = npad // NS
    cbh = cb // 2
    zrows = jnp.zeros((npad, cc), F32)
    src2 = src.reshape(-1, EK)
    dst2 = dst.reshape(-1, EK)
    c2 = chunks - (chunks % 2)

    @functools.partial(
        pl.kernel,
        out_type=jax.ShapeDtypeStruct((cb, npad, cc), F32),
        mesh=_vmesh(),
        compiler_params=pltpu.CompilerParams(use_tc_tiling_on_sc=False),
        scratch_types=[
            pltpu.VMEM((chunks, EK), jnp.int32),
            pltpu.VMEM((chunks, EK), jnp.int32),
            pltpu.VMEM((chunks, EK), jnp.int32),
            pltpu.VMEM((EK, cc), F32),
            pltpu.VMEM((EK, cc), F32),
            pltpu.VMEM_SHARED((npad, cc), F32),
            pltpu.SemaphoreType.DMA,
            pltpu.SemaphoreType.DMA,
        ],
    )
    def k(hp_hbm, src_hbm, dst_hbm, z_hbm, out_hbm, src2d, off2d, dst2d,
          rowa, rowb, acc, sema, semb):
        cid = lax.axis_index("c")
        sid = lax.axis_index("s")
        cbase = sid * chunks
        pltpu.sync_copy(src_hbm.at[pl.ds(cbase, chunks)], src2d)
        pltpu.sync_copy(dst_hbm.at[pl.ds(cbase, chunks)], dst2d)

        def wait_g(buf, sem):
            pltpu.make_async_copy(hp_hbm.at[pl.ds(0, EK)], buf, sem).wait()

        def scat(jj, buf):
            pltpu.sync_copy(buf, acc.at[dst2d.at[jj]], add=True)

        for lb in range(cbh):
            b = cid * cbh + lb

            @pl.loop(0, chunks)
            def _(j):
                @pl.loop(0, EK, step=16)
                def _(t):
                    off2d[j, pl.ds(t, 16)] = src2d[j, pl.ds(t, 16)] + b * n

            pltpu.sync_copy(z_hbm.at[pl.ds(sid * rp, rp)],
                            acc.at[pl.ds(sid * rp, rp)])
            plsc.subcore_barrier()

            def start_g(jj, buf, sem):
                pltpu.async_copy(hp_hbm.at[off2d.at[jj]], buf, sem)

            if c2 > 0:
                start_g(0, rowa, sema)

                @pl.loop(0, c2, step=2)
                def _(j):
                    start_g(j + 1, rowb, semb)
                    wait_g(rowa, sema)
                    scat(j, rowa)

                    @pl.when(j + 2 < c2)
                    def _():
                        start_g(j + 2, rowa, sema)

                    wait_g(rowb, semb)
                    scat(j + 1, rowb)

            if chunks % 2:
                start_g(chunks - 1, rowa, sema)
                wait_g(rowa, sema)
                scat(chunks - 1, rowa)

            plsc.subcore_barrier()
            pltpu.sync_copy(acc.at[pl.ds(sid * rp, rp)],
                            out_hbm.at[b].at[pl.ds(sid * rp, rp)])
            plsc.subcore_barrier()

    return k(hp, src2, dst2, zrows)


def _tc_post_split(seg, h, dis, bias, cb):
    m, c = h.shape
    cc = c // cb

    def body(s_ref, h_ref, d_ref, b_ref, o_ref):
        s = s_ref[0]
        d = d_ref[:, :1]
        o_ref[...] = jax.nn.relu(d * s + d * d * h_ref[...] + b_ref[...])

    grid = (pl.cdiv(m, MB), cb)
    return pl.pallas_call(
        body,
        grid=grid,
        in_specs=[
            pl.BlockSpec((1, MB, cc), lambda i, b: (b, i, 0)),
            pl.BlockSpec((MB, cc), lambda i, b: (i, b)),
            pl.BlockSpec((MB, 16), lambda i, b: (i, 0)),
            pl.BlockSpec((1, cc), lambda i, b: (0, b)),
        ],
        out_specs=pl.BlockSpec((MB, cc), lambda i, b: (i, b)),
        out_shape=jax.ShapeDtypeStruct((m, c), F32),
    )(seg, h, dis, bias)


def _sc_gather_wide(table, idx, cc):
    if cc <= 256:
        return _sc_gather(table, idx, cc)
    s = cc // 256
    mp = idx.shape[0]
    idxe = (idx[:, None] * s + jnp.arange(s, dtype=jnp.int32)[None, :]).reshape(-1)
    out = _sc_gather(table.reshape(table.shape[0] * s, 256), idxe, 256)
    return out.reshape(mp, cc)


def _sc_gather(table, idx, cc):
    mp = idx.shape[0]
    per_w = mp // NW
    chunks = per_w // EK

    @functools.partial(
        pl.kernel,
        out_type=jax.ShapeDtypeStruct((mp, cc), F32),
        mesh=_vmesh(),
        compiler_params=pltpu.CompilerParams(use_tc_tiling_on_sc=False),
        scratch_types=[
            pltpu.VMEM((chunks, EK), jnp.int32),
            pltpu.VMEM((EK, cc), F32),
            pltpu.VMEM((EK, cc), F32),
            pltpu.SemaphoreType.DMA,
            pltpu.SemaphoreType.DMA,
        ],
    )
    def k(t_hbm, i_hbm, o_hbm, idx2d, rowa, rowb, sema, semb):
        cid = lax.axis_index("c")
        sid = lax.axis_index("s")
        wid = cid * NS + sid
        base = wid * per_w
        cbase = wid * chunks
        pltpu.sync_copy(i_hbm.at[pl.ds(cbase, chunks)], idx2d)

        def start_g(jj, buf, sem):
            pltpu.async_copy(t_hbm.at[idx2d.at[jj]], buf, sem)

        def wait_g(buf, sem):
            pltpu.make_async_copy(t_hbm.at[pl.ds(0, EK)], buf, sem).wait()

        def store(jj, buf):
            pltpu.sync_copy(buf, o_hbm.at[pl.ds(base + jj * EK, EK)])

        c2 = chunks - (chunks % 2)
        if c2 > 0:
            start_g(0, rowa, sema)

            @pl.loop(0, c2, step=2)
            def _(j):
                start_g(j + 1, rowb, semb)
                wait_g(rowa, sema)
                store(j, rowa)

                @pl.when(j + 2 < c2)
                def _():
                    start_g(j + 2, rowa, sema)

                wait_g(rowb, semb)
                store(j + 1, rowb)

        if chunks % 2:
            start_g(chunks - 1, rowa, sema)
            wait_g(rowa, sema)
            store(chunks - 1, rowa)

    return k(table, idx.reshape(-1, EK))


def _tc_dis(cnt, n):
    def body(c_ref, o_ref):
        c = c_ref[0] + c_ref[1]
        o_ref[...] = lax.rsqrt(1.0 + c)

    grid = (pl.cdiv(n, MB),)
    return pl.pallas_call(
        body,
        grid=grid,
        in_specs=[pl.BlockSpec((NC, MB, 16), lambda m: (0, m, 0))],
        out_specs=pl.BlockSpec((MB, 16), lambda m: (m, 0)),
        out_shape=jax.ShapeDtypeStruct((n, 16), F32),
    )(cnt)


def _tc_mm(a, w, dis, cb):
    m, kk = a.shape
    c = w.shape[1]
    cc = c // cb

    def body(a_ref, w_ref, d_ref, h_ref, hp_ref):
        h = jnp.dot(a_ref[...], w_ref[...], preferred_element_type=F32,
                    precision=HIGH)
        h_ref[...] = h
        hp_ref[0] = d_ref[:, :1] * h

    grid = (pl.cdiv(m, MB), cb)
    return pl.pallas_call(
        body,
        grid=grid,
        in_specs=[
            pl.BlockSpec((MB, kk), lambda i, b: (i, 0)),
            pl.BlockSpec((kk, cc), lambda i, b: (0, b)),
            pl.BlockSpec((MB, 16), lambda i, b: (i, 0)),
        ],
        out_specs=[
            pl.BlockSpec((MB, cc), lambda i, b: (i, b)),
            pl.BlockSpec((1, MB, cc), lambda i, b: (b, i, 0)),
        ],
        out_shape=[
            jax.ShapeDtypeStruct((m, c), F32),
            jax.ShapeDtypeStruct((cb, m, cc), F32),
        ],
    )(a, w, dis)


def _tc_mm_fp2(a, w_rest, w_pool, pooled, dis, cb):
    m, kk = a.shape
    c = w_rest.shape[1]
    cc = c // cb

    def body(a_ref, wr_ref, wp_ref, p_ref, d_ref, h_ref, hp_ref):
        prow = jnp.dot(p_ref[:1], wp_ref[...], preferred_element_type=F32,
                       precision=HIGH)
        h = jnp.dot(a_ref[...], wr_ref[...], preferred_element_type=F32,
                    precision=HIGH) + prow
        h_ref[...] = h
        hp_ref[0] = d_ref[:, :1] * h

    grid = (pl.cdiv(m, MB), cb)
    gtd = w_pool.shape[0]
    return pl.pallas_call(
        body,
        grid=grid,
        in_specs=[
            pl.BlockSpec((MB, kk), lambda i, b: (i, 0)),
            pl.BlockSpec((kk, cc), lambda i, b: (0, b)),
            pl.BlockSpec((gtd, cc), lambda i, b: (0, b)),
            pl.BlockSpec((8, gtd), lambda i, b: (0, 0)),
            pl.BlockSpec((MB, 16), lambda i, b: (i, 0)),
        ],
        out_specs=[
            pl.BlockSpec((MB, cc), lambda i, b: (i, b)),
            pl.BlockSpec((1, MB, cc), lambda i, b: (b, i, 0)),
        ],
        out_shape=[
            jax.ShapeDtypeStruct((m, c), F32),
            jax.ShapeDtypeStruct((cb, m, cc), F32),
        ],
    )(a, w_rest, w_pool, pooled, dis)


def _tc_post(seg, h, dis, bias, cb):
    m, c = h.shape
    cc = c // cb

    def body(s_ref, h_ref, d_ref, b_ref, o_ref):
        s = s_ref[0, 0] + s_ref[1, 0]
        d = d_ref[:, :1]
        o_ref[...] = jax.nn.relu(d * s + d * d * h_ref[...] + b_ref[...])

    grid = (pl.cdiv(m, MB), cb)
    return pl.pallas_call(
        body,
        grid=grid,
        in_specs=[
            pl.BlockSpec((NC, 1, MB, cc), lambda i, b: (0, b, i, 0)),
            pl.BlockSpec((MB, cc), lambda i, b: (i, b)),
            pl.BlockSpec((MB, 16), lambda i, b: (i, 0)),
            pl.BlockSpec((1, cc), lambda i, b: (0, b)),
        ],
        out_specs=pl.BlockSpec((MB, cc), lambda i, b: (i, b)),
        out_shape=jax.ShapeDtypeStruct((m, c), F32),
    )(seg, h, dis, bias)


def _tc_knn3(pos_y, pos_x):
    ny = pos_y.shape[0]
    nx = pos_x.shape[0]

    def body(py_ref, px_ref, i_ref, w_ref):
        py = py_ref[...]
        px = px_ref[...]
        py2 = jnp.sum(py * py, axis=1, keepdims=True)
        px2 = lax.dot_general(jnp.ones((1, 3), F32), px * px,
                              (((1,), (1,)), ((), ())),
                              preferred_element_type=F32, precision=HIGH)
        cross = lax.dot_general(py, px, (((1,), (1,)), ((), ())),
                                preferred_element_type=F32, precision=HIGH)
        d = py2 - 2.0 * cross + px2
        col = lax.broadcasted_iota(jnp.int32, (MB, nx), 1)
        idxs = []
        ws = []
        for _ in range(3):
            mv = jnp.min(d, axis=1, keepdims=True)
            am = jnp.min(jnp.where(d == mv, col, nx), axis=1, keepdims=True)
            idxs.append(am)
            ws.append(1.0 / jnp.maximum(mv, 1e-16))
            d = jnp.where(col == am, 1e30, d)
        i_ref[...] = jnp.concatenate(idxs, axis=1)
        wst = jnp.concatenate(ws, axis=1)
        w_ref[...] = wst / jnp.sum(wst, axis=1, keepdims=True)

    grid = (pl.cdiv(ny, MB),)
    return pl.pallas_call(
        body,
        grid=grid,
        in_specs=[
            pl.BlockSpec((MB, 3), lambda i: (i, 0)),
            pl.BlockSpec((nx, 3), lambda i: (0, 0)),
        ],
        out_specs=[
            pl.BlockSpec((MB, 3), lambda i: (i, 0)),
            pl.BlockSpec((MB, 3), lambda i: (i, 0)),
        ],
        out_shape=[
            jax.ShapeDtypeStruct((ny, 3), jnp.int32),
            jax.ShapeDtypeStruct((ny, 3), F32),
        ],
    )(pos_y, pos_x)


def _tc_wsum(feats, wn):
    _, ny, cc = feats.shape

    def body(f_ref, w_ref, o_ref):
        o_ref[...] = (f_ref[0] * w_ref[:, 0:1] + f_ref[1] * w_ref[:, 1:2]
                      + f_ref[2] * w_ref[:, 2:3])

    grid = (pl.cdiv(ny, MB),)
    return pl.pallas_call(
        body,
        grid=grid,
        in_specs=[
            pl.BlockSpec((3, MB, cc), lambda i: (0, i, 0)),
            pl.BlockSpec((MB, 3), lambda i: (i, 0)),
        ],
        out_specs=pl.BlockSpec((MB, cc), lambda i: (i, 0)),
        out_shape=jax.ShapeDtypeStruct((ny, cc), F32),
    )(feats, wn)


def _tc_mlp_gp(x2pos, lins, bns):
    (w1, b1), (w2, b2), (w3, b3) = lins
    (g1, e1), (g2, e2) = bns
    gtd = w3.shape[1]

    def body(x_ref, w1_ref, b1_ref, g1_ref, e1_ref, w2_ref, b2_ref, g2_ref,
             e2_ref, w3_ref, b3_ref, o_ref):
        def bn_relu(h, g_ref, e_ref):
            mu = jnp.mean(h, axis=0, keepdims=True)
            var = jnp.mean((h - mu) ** 2, axis=0, keepdims=True)
            h = (h - mu) / jnp.sqrt(var + 1e-5) * g_ref[...] + e_ref[...]
            return jax.nn.relu(h)

        h = jnp.dot(x_ref[...], w1_ref[...], preferred_element_type=F32,
                    precision=HIGH) + b1_ref[...]
        h = bn_relu(h, g1_ref, e1_ref)
        h = jnp.dot(h, w2_ref[...], preferred_element_type=F32,
                    precision=HIGH) + b2_ref[...]
        h = bn_relu(h, g2_ref, e2_ref)
        h = jnp.dot(h, w3_ref[...], preferred_element_type=F32,
                    precision=HIGH) + b3_ref[...]
        pooled = jnp.max(h, axis=0, keepdims=True)
        o_ref[...] = jnp.broadcast_to(pooled, (8, gtd))

    args = (x2pos, w1, b1.reshape(1, -1), g1.reshape(1, -1), e1.reshape(1, -1),
            w2, b2.reshape(1, -1), g2.reshape(1, -1), e2.reshape(1, -1),
            w3, b3.reshape(1, -1))
    return pl.pallas_call(
        body,
        out_shape=jax.ShapeDtypeStruct((8, gtd), F32),
    )(*args)


def _tc_head(h, w1, b1, w2, b2):
    m = h.shape[0]
    co = w2.shape[1]

    def body(h_ref, w1_ref, b1_ref, w2_ref, b2_ref, o_ref):
        t = jax.nn.relu(jnp.dot(h_ref[...], w1_ref[...],
                                preferred_element_type=F32, precision=HIGH)
                        + b1_ref[...])
        o_ref[...] = jnp.dot(t, w2_ref[...], preferred_element_type=F32,
                             precision=HIGH) + b2_ref[...]

    grid = (pl.cdiv(m, MB),)
    return pl.pallas_call(
        body,
        grid=grid,
        in_specs=[
            pl.BlockSpec((MB, w1.shape[0]), lambda i: (i, 0)),
            pl.BlockSpec(w1.shape, lambda i: (0, 0)),
            pl.BlockSpec((1, w1.shape[1]), lambda i: (0, 0)),
            pl.BlockSpec(w2.shape, lambda i: (0, 0)),
            pl.BlockSpec((1, co), lambda i: (0, 0)),
        ],
        out_specs=pl.BlockSpec((MB, co), lambda i: (i, 0)),
        out_shape=jax.ShapeDtypeStruct((m, co), F32),
    )(h, w1, b1.reshape(1, -1), w2, b2.reshape(1, -1))


def _pad_edges(ei, n):
    e = ei.shape[1]
    ep = _rup(e, NW * EK)
    src = jnp.concatenate([ei[0].astype(jnp.int32),
                           jnp.zeros((ep - e,), jnp.int32)])
    dst = jnp.concatenate([ei[1].astype(jnp.int32),
                           jnp.full((ep - e,), n, jnp.int32)])
    return src, dst


def _pad_idx(idx):
    mpad = _rup(idx.shape[0], NW * EK)
    return jnp.concatenate([idx.astype(jnp.int32),
                            jnp.zeros((mpad - idx.shape[0],), jnp.int32)])


def _pad_cols(a, cc):
    if a.shape[1] == cc:
        return a
    return jnp.concatenate(
        [a, jnp.zeros((a.shape[0], cc - a.shape[1]), F32)], axis=1)


def _conv_block(a, convs, src, dst, dis, n, cbs):
    h_act = a
    for (w, b), cb in zip(convs, cbs):
        c = w.shape[1]
        cc = c // cb
        h, hp = _tc_mm(h_act, w, dis, cb)
        if cb % 2 == 0 and _npad(n) * cc <= 900000:
            seg = _sc_conv_scatter_split(hp.reshape(cb * n, cc), src, dst,
                                         n, cb, cc)
            h_act = _tc_post_split(seg, h, dis, b.reshape(1, -1), cb)
        else:
            seg = _sc_conv_scatter(hp.reshape(cb * n, cc), src, dst, n, cb, cc)
            h_act = _tc_post(seg, h, dis, b.reshape(1, -1), cb)
    return h_act


def kernel(x, pos, params, batch, idx0, idx1, edge_index0, edge_index1,
           edge_index2):
    n0 = x.shape[0]
    n1 = idx0.shape[0]
    n2 = idx1.shape[0]
    in_c = x.shape[1]

    src0, dst0 = _pad_edges(edge_index0, n0)
    src1, dst1 = _pad_edges(edge_index1, n1)
    src2, dst2 = _pad_edges(edge_index2, n2)

    dis0 = _tc_dis(_sc_degree(dst0, n0), n0)
    dis1 = _tc_dis(_sc_degree(dst1, n1), n1)
    dis2 = _tc_dis(_sc_degree(dst2, n2), n2)

    h = _conv_block(x, params['sa1'], src0, dst0, dis0, n0, cbs=[1, 1, 1])

    tbl = _pad_cols(jnp.concatenate([h, pos], axis=1), 80)
    g1 = _sc_gather(tbl, _pad_idx(idx0), 80)[:n1]
    x1 = g1[:, :h.shape[1]]
    pos1 = g1[:, h.shape[1]:h.shape[1] + 3]
    a1 = g1[:, :h.shape[1] + 3]

    h = _conv_block(a1, params['sa2'], src1, dst1, dis1, n1, cbs=[1, 1, 1])

    tbl = _pad_cols(jnp.concatenate([h, pos1], axis=1), 144)
    g2 = _sc_gather(tbl, _pad_idx(idx1), 144)[:n2]
    x2pos = g2[:, :h.shape[1] + 3]
    pos2 = g2[:, h.shape[1]:h.shape[1] + 3]

    xb = _conv_block(x2pos, params['bn'], src2, dst2, dis2, n2, cbs=[1, 1, 1])

    pooled = _tc_mlp_gp(x2pos, params['gp_lin'], params['gp_bn'])

    idxk, wn = _tc_knn3(pos1, pos2)
    flat = _pad_idx(jnp.transpose(idxk).reshape(-1))
    feats = _sc_gather_wide(xb, flat, xb.shape[1])[:3 * n1].reshape(3, n1, -1)
    interp_xb = _tc_wsum(feats, wn)
    a_small = jnp.concatenate([interp_xb, x1], axis=1)

    (w, b) = params['fp2'][0]
    gtd = pooled.shape[1]
    h, hp = _tc_mm_fp2(a_small, w[gtd:], w[:gtd], pooled, dis1, cb=4)
    cc = w.shape[1] // 4
    seg = _sc_conv_scatter_split(hp.reshape(4 * n1, cc), src1, dst1, n1, 4, cc)
    h = _tc_post_split(seg, h, dis1, b.reshape(1, -1), 4)
    h = _conv_block(h, params['fp2'][1:], src1, dst1, dis1, n1, cbs=[4, 2])

    idxk, wn = _tc_knn3(pos, pos1)
    flat = _pad_idx(jnp.transpose(idxk).reshape(-1))
    feats = _sc_gather_wide(h, flat, h.shape[1])[:3 * n0].reshape(3, n0, -1)
    interp = _tc_wsum(feats, wn)
    a0 = jnp.concatenate([interp, x[:, :in_c]], axis=1)

    h = _conv_block(a0, params['fp1'], src0, dst0, dis0, n0, cbs=[2, 2, 1])

    (w1, b1), (w2, b2) = params['head']
    return _tc_head(h, w1, b1, w2, b2)


# speedup vs baseline: 1.0920x; 1.0017x over previous
import functools
import jax
import jax.numpy as jnp
from jax import lax
from jax.experimental import pallas as pl
from jax.experimental.pallas import tpu as pltpu
from jax.experimental.pallas import tpu_sc as plsc

F32 = jnp.float32
NC, NS = 2, 16
NW = NC * NS
EK = 128
MB = 512
HIGH = lax.Precision.HIGHEST

_vmesh_cache = []


def _vmesh():
    if not _vmesh_cache:
        _vmesh_cache.append(
            plsc.VectorSubcoreMesh(core_axis_name="c", subcore_axis_name="s"))
    return _vmesh_cache[0]


def _rup(v, m):
    return ((v + m - 1) // m) * m


def _npad(n):
    return _rup(n + 1, 128)


def _sc_degree(dst, n):
    npad = _npad(n)
    ep = dst.shape[0]
    chunks = ep // (NW * EK)
    rp = npad // NS
    ones = jnp.ones((EK, 16), F32)
    zrows = jnp.zeros((npad, 16), F32)

    @functools.partial(
        pl.kernel,
        out_type=jax.ShapeDtypeStruct((NC, npad, 16), F32),
        mesh=_vmesh(),
        compiler_params=pltpu.CompilerParams(use_tc_tiling_on_sc=False),
        scratch_types=[
            pltpu.VMEM((chunks, EK), jnp.int32),
            pltpu.VMEM((EK, 16), F32),
            pltpu.VMEM_SHARED((npad, 16), F32),
            pltpu.SemaphoreType.DMA,
        ],
    )
    def k(dst_hbm, ones_hbm, z_hbm, out_hbm, dst2d, onesv, acc, sem):
        cid = lax.axis_index("c")
        sid = lax.axis_index("s")
        wid = cid * NS + sid
        pltpu.sync_copy(dst_hbm.at[pl.ds(wid * chunks, chunks)], dst2d)
        pltpu.sync_copy(ones_hbm, onesv)
        pltpu.sync_copy(z_hbm.at[pl.ds(sid * rp, rp)], acc.at[pl.ds(sid * rp, rp)])
        plsc.subcore_barrier()

        for j0 in range(0, chunks, 8):
            jw = min(8, chunks - j0)
            for j in range(j0, j0 + jw):
                pltpu.async_copy(onesv, acc.at[dst2d.at[j]], sem, add=True)
            for _ in range(jw):
                pltpu.make_async_copy(onesv, acc.at[dst2d.at[0]], sem).wait()

        plsc.subcore_barrier()
        pltpu.sync_copy(acc.at[pl.ds(sid * rp, rp)],
                        out_hbm.at[cid].at[pl.ds(sid * rp, rp)])

    return k(dst.reshape(-1, EK), ones, zrows)


def _sc_conv_scatter(hp, src, dst, n, cb, cc):
    npad = _npad(n)
    ep = src.shape[0]
    chunks = ep // (NW * EK)
    rp = npad // NS
    zrows = jnp.zeros((npad, cc), F32)
    src2 = src.reshape(-1, EK)
    dst2 = dst.reshape(-1, EK)
    c2 = chunks - (chunks % 2)

    @functools.partial(
        pl.kernel,
        out_type=jax.ShapeDtypeStruct((NC, cb, npad, cc), F32),
        mesh=_vmesh(),
        compiler_params=pltpu.CompilerParams(use_tc_tiling_on_sc=False),
        scratch_types=[
            pltpu.VMEM((chunks, EK), jnp.int32),
            pltpu.VMEM((chunks, EK), jnp.int32),
            pltpu.VMEM((chunks, EK), jnp.int32),
            pltpu.VMEM((EK, cc), F32),
            pltpu.VMEM((EK, cc), F32),
            pltpu.VMEM_SHARED((npad, cc), F32),
            pltpu.SemaphoreType.DMA,
            pltpu.SemaphoreType.DMA,
        ],
    )
    def k(hp_hbm, src_hbm, dst_hbm, z_hbm, out_hbm, src2d, off2d, dst2d,
          rowa, rowb, acc, sema, semb):
        cid = lax.axis_index("c")
        sid = lax.axis_index("s")
        wid = cid * NS + sid
        cbase = wid * chunks
        pltpu.sync_copy(src_hbm.at[pl.ds(cbase, chunks)], src2d)
        pltpu.sync_copy(dst_hbm.at[pl.ds(cbase, chunks)], dst2d)

        def wait_g(buf, sem):
            pltpu.make_async_copy(hp_hbm.at[pl.ds(0, EK)], buf, sem).wait()

        def scat(jj, buf):
            pltpu.sync_copy(buf, acc.at[dst2d.at[jj]], add=True)

        for b in range(cb):
            idx2 = src2d if b == 0 else off2d
            if b > 0:
                @pl.loop(0, chunks)
                def _(j):
                    @pl.loop(0, EK, step=16)
                    def _(t):
                        off2d[j, pl.ds(t, 16)] = src2d[j, pl.ds(t, 16)] + b * n

            pltpu.sync_copy(z_hbm.at[pl.ds(sid * rp, rp)],
                            acc.at[pl.ds(sid * rp, rp)])
            plsc.subcore_barrier()

            def start_g(jj, buf, sem, idx2=idx2):
                pltpu.async_copy(hp_hbm.at[idx2.at[jj]], buf, sem)

            if c2 > 0:
                start_g(0, rowa, sema)

                @pl.loop(0, c2, step=2)
                def _(j):
                    start_g(j + 1, rowb, semb)
                    wait_g(rowa, sema)
                    scat(j, rowa)

                    @pl.when(j + 2 < c2)
                    def _():
                        start_g(j + 2, rowa, sema)

                    wait_g(rowb, semb)
                    scat(j + 1, rowb)

            if chunks % 2:
                start_g(chunks - 1, rowa, sema)
                wait_g(rowa, sema)
                scat(chunks - 1, rowa)

            plsc.subcore_barrier()
            pltpu.sync_copy(acc.at[pl.ds(sid * rp, rp)],
                            out_hbm.at[cid].at[b].at[pl.ds(sid * rp, rp)])
            plsc.subcore_barrier()

    return k(hp, src2, dst2, zrows)


def _sc_conv_scatter_split(hp, src, dst, n, cb, cc):
    npad = _npad(n)
    ep = src.shape[0]
    chunks = ep // (NS * EK)
    rp = npad // NS
    cbh = cb // 2
    zrows = jnp.zeros((npad, cc), F32)
    src2 = src.reshape(-1, EK)
    dst2 = dst.reshape(-1, EK)
    c2 = chunks - (chunks % 2)

    @functools.partial(
        pl.kernel,
        out_type=jax.ShapeDtypeStruct((cb, npad, cc), F32),
        mesh=_vmesh(),
        compiler_params=pltpu.CompilerParams(use_tc_tiling_on_sc=False),
        scratch_types=[
            pltpu.VMEM((chunks, EK), jnp.int32),
            pltpu.VMEM((chunks, EK), jnp.int32),
            pltpu.VMEM((chunks, EK), jnp.int32),
            pltpu.VMEM((EK, cc), F32),
            pltpu.VMEM((EK, cc), F32),
            pltpu.VMEM_SHARED((npad, cc), F32),
            pltpu.SemaphoreType.DMA,
            pltpu.SemaphoreType.DMA,
        ],
    )
    def k(hp_hbm, src_hbm, dst_hbm, z_hbm, out_hbm, src2d, off2d, dst2d,
          rowa, rowb, acc, sema, semb):
        cid = lax.axis_index("c")
        sid = lax.axis_index("s")
        cbase = sid * chunks
        pltpu.sync_copy(src_hbm.at[pl.ds(cbase, chunks)], src2d)
        pltpu.sync_copy(dst_hbm.at[pl.ds(cbase, chunks)], dst2d)

        def wait_g(buf, sem):
            pltpu.make_async_copy(hp_hbm.at[pl.ds(0, EK)], buf, sem).wait()

        def scat(jj, buf):
            pltpu.sync_copy(buf, acc.at[dst2d.at[jj]], add=True)

        for lb in range(cbh):
            b = cid * cbh + lb

            @pl.loop(0, chunks)
            def _(j):
                @pl.loop(0, EK, step=16)
                def _(t):
                    off2d[j, pl.ds(t, 16)] = src2d[j, pl.ds(t, 16)] + b * n

            pltpu.sync_copy(z_hbm.at[pl.ds(sid * rp, rp)],
                            acc.at[pl.ds(sid * rp, rp)])
            plsc.subcore_barrier()

            def start_g(jj, buf, sem):
                pltpu.async_copy(hp_hbm.at[off2d.at[jj]], buf, sem)

            if c2 > 0:
                start_g(0, rowa, sema)

                @pl.loop(0, c2, step=2)
                def _(j):
                    start_g(j + 1, rowb, semb)
                    wait_g(rowa, sema)
                    scat(j, rowa)

                    @pl.when(j + 2 < c2)
                    def _():
                        start_g(j + 2, rowa, sema)

                    wait_g(rowb, semb)
                    scat(j + 1, rowb)

            if chunks % 2:
                start_g(chunks - 1, rowa, sema)
                wait_g(rowa, sema)
                scat(chunks - 1, rowa)

            plsc.subcore_barrier()
            pltpu.sync_copy(acc.at[pl.ds(sid * rp, rp)],
                            out_hbm.at[b].at[pl.ds(sid * rp, rp)])
            plsc.subcore_barrier()

    return k(hp, src2, dst2, zrows)


def _tc_post_split(seg, h, dis, bias, cb):
    m, c = h.shape
    cc = c // cb

    def body(s_ref, h_ref, d_ref, b_ref, o_ref):
        s = s_ref[0]
        d = d_ref[:, :1]
        o_ref[...] = jax.nn.relu(d * s + d * d * h_ref[...] + b_ref[...])

    grid = (pl.cdiv(m, MB), cb)
    return pl.pallas_call(
        body,
        grid=grid,
        in_specs=[
            pl.BlockSpec((1, MB, cc), lambda i, b: (b, i, 0)),
            pl.BlockSpec((MB, cc), lambda i, b: (i, b)),
            pl.BlockSpec((MB, 16), lambda i, b: (i, 0)),
            pl.BlockSpec((1, cc), lambda i, b: (0, b)),
        ],
        out_specs=pl.BlockSpec((MB, cc), lambda i, b: (i, b)),
        out_shape=jax.ShapeDtypeStruct((m, c), F32),
    )(seg, h, dis, bias)


def _sc_conv_scatter4(hp, src, dst, n, cc):
    npad = _npad(n)
    ep = src.shape[0]
    chunks = ep // (NW * EK)
    rp = npad // NS
    zrows = jnp.zeros((npad, cc), F32)
    src2 = src.reshape(-1, EK)
    dst2 = dst.reshape(-1, EK)
    nbb = min(4, chunks)
    c4 = (chunks // 4) * 4

    @functools.partial(
        pl.kernel,
        out_type=jax.ShapeDtypeStruct((NC, 1, npad, cc), F32),
        mesh=_vmesh(),
        compiler_params=pltpu.CompilerParams(use_tc_tiling_on_sc=False),
        scratch_types=(
            [pltpu.VMEM((chunks, EK), jnp.int32)] * 2
            + [pltpu.VMEM((EK, cc), F32)] * nbb
            + [pltpu.VMEM_SHARED((npad, cc), F32)]
            + [pltpu.SemaphoreType.DMA] * nbb
        ),
    )
    def k(hp_hbm, src_hbm, dst_hbm, z_hbm, out_hbm, src2d, dst2d, *rest):
        rows = rest[:nbb]
        acc = rest[nbb]
        sems = rest[nbb + 1:]
        cid = lax.axis_index("c")
        sid = lax.axis_index("s")
        wid = cid * NS + sid
        cbase = wid * chunks
        pltpu.sync_copy(src_hbm.at[pl.ds(cbase, chunks)], src2d)
        pltpu.sync_copy(dst_hbm.at[pl.ds(cbase, chunks)], dst2d)
        pltpu.sync_copy(z_hbm.at[pl.ds(sid * rp, rp)],
                        acc.at[pl.ds(sid * rp, rp)])
        plsc.subcore_barrier()

        def start_g(jj, r):
            pltpu.async_copy(hp_hbm.at[src2d.at[jj]], rows[r], sems[r])

        def wait_g(r):
            pltpu.make_async_copy(hp_hbm.at[pl.ds(0, EK)], rows[r],
                                  sems[r]).wait()

        def scat(jj, r):
            pltpu.sync_copy(rows[r], acc.at[dst2d.at[jj]], add=True)

        for r in range(nbb):
            start_g(r, r)
        if c4 >= 4:
            @pl.loop(0, c4, step=4)
            def _(j):
                for off in range(4):
                    wait_g(off)
                    scat(j + off, off)

                    @pl.when(j + off + 4 < chunks)
                    def _():
                        start_g(j + off + 4, off)

        for jj in range(c4, chunks):
            wait_g(jj % 4 if c4 >= 4 else jj)
            scat(jj, jj % 4 if c4 >= 4 else jj)

        plsc.subcore_barrier()
        pltpu.sync_copy(acc.at[pl.ds(sid * rp, rp)],
                        out_hbm.at[cid].at[0].at[pl.ds(sid * rp, rp)])

    return k(hp, src2, dst2, zrows)


def _sc_gather_wide(table, idx, cc):
    if cc <= 256:
        return _sc_gather(table, idx, cc)
    s = cc // 256
    mp = idx.shape[0]
    idxe = (idx[:, None] * s + jnp.arange(s, dtype=jnp.int32)[None, :]).reshape(-1)
    out = _sc_gather(table.reshape(table.shape[0] * s, 256), idxe, 256)
    return out.reshape(mp, cc)


def _sc_gather(table, idx, cc):
    mp = idx.shape[0]
    per_w = mp // NW
    chunks = per_w // EK

    @functools.partial(
        pl.kernel,
        out_type=jax.ShapeDtypeStruct((mp, cc), F32),
        mesh=_vmesh(),
        compiler_params=pltpu.CompilerParams(use_tc_tiling_on_sc=False),
        scratch_types=[
            pltpu.VMEM((chunks, EK), jnp.int32),
            pltpu.VMEM((EK, cc), F32),
            pltpu.VMEM((EK, cc), F32),
            pltpu.SemaphoreType.DMA,
            pltpu.SemaphoreType.DMA,
        ],
    )
    def k(t_hbm, i_hbm, o_hbm, idx2d, rowa, rowb, sema, semb):
        cid = lax.axis_index("c")
        sid = lax.axis_index("s")
        wid = cid * NS + sid
        base = wid * per_w
        cbase = wid * chunks
        pltpu.sync_copy(i_hbm.at[pl.ds(cbase, chunks)], idx2d)

        def start_g(jj, buf, sem):
            pltpu.async_copy(t_hbm.at[idx2d.at[jj]], buf, sem)

        def wait_g(buf, sem):
            pltpu.make_async_copy(t_hbm.at[pl.ds(0, EK)], buf, sem).wait()

        def store(jj, buf):
            pltpu.sync_copy(buf, o_hbm.at[pl.ds(base + jj * EK, EK)])

        c2 = chunks - (chunks % 2)
        if c2 > 0:
            start_g(0, rowa, sema)

            @pl.loop(0, c2, step=2)
            def _(j):
                start_g(j + 1, rowb, semb)
                wait_g(rowa, sema)
                store(j, rowa)

                @pl.when(j + 2 < c2)
                def _():
                    start_g(j + 2, rowa, sema)

                wait_g(rowb, semb)
                store(j + 1, rowb)

        if chunks % 2:
            start_g(chunks - 1, rowa, sema)
            wait_g(rowa, sema)
            store(chunks - 1, rowa)

    return k(table, idx.reshape(-1, EK))


def _tc_dis(cnt, n):
    def body(c_ref, o_ref):
        c = c_ref[0] + c_ref[1]
        o_ref[...] = lax.rsqrt(1.0 + c)

    grid = (pl.cdiv(n, MB),)
    return pl.pallas_call(
        body,
        grid=grid,
        in_specs=[pl.BlockSpec((NC, MB, 16), lambda m: (0, m, 0))],
        out_specs=pl.BlockSpec((MB, 16), lambda m: (m, 0)),
        out_shape=jax.ShapeDtypeStruct((n, 16), F32),
    )(cnt)


def _tc_mm(a, w, dis, cb):
    m, kk = a.shape
    c = w.shape[1]
    cc = c // cb

    def body(a_ref, w_ref, d_ref, h_ref, hp_ref):
        h = jnp.dot(a_ref[...], w_ref[...], preferred_element_type=F32,
                    precision=HIGH)
        h_ref[...] = h
        hp_ref[0] = d_ref[:, :1] * h

    grid = (pl.cdiv(m, MB), cb)
    return pl.pallas_call(
        body,
        grid=grid,
        in_specs=[
            pl.BlockSpec((MB, kk), lambda i, b: (i, 0)),
            pl.BlockSpec((kk, cc), lambda i, b: (0, b)),
            pl.BlockSpec((MB, 16), lambda i, b: (i, 0)),
        ],
        out_specs=[
            pl.BlockSpec((MB, cc), lambda i, b: (i, b)),
            pl.BlockSpec((1, MB, cc), lambda i, b: (b, i, 0)),
        ],
        out_shape=[
            jax.ShapeDtypeStruct((m, c), F32),
            jax.ShapeDtypeStruct((cb, m, cc), F32),
        ],
    )(a, w, dis)


def _tc_mm_fp2(a, w_rest, w_pool, pooled, dis, cb):
    m, kk = a.shape
    c = w_rest.shape[1]
    cc = c // cb

    def body(a_ref, wr_ref, wp_ref, p_ref, d_ref, h_ref, hp_ref):
        prow = jnp.dot(p_ref[:1], wp_ref[...], preferred_element_type=F32,
                       precision=HIGH)
        h = jnp.dot(a_ref[...], wr_ref[...], preferred_element_type=F32,
                    precision=HIGH) + prow
        h_ref[...] = h
        hp_ref[0] = d_ref[:, :1] * h

    grid = (pl.cdiv(m, MB), cb)
    gtd = w_pool.shape[0]
    return pl.pallas_call(
        body,
        grid=grid,
        in_specs=[
            pl.BlockSpec((MB, kk), lambda i, b: (i, 0)),
            pl.BlockSpec((kk, cc), lambda i, b: (0, b)),
            pl.BlockSpec((gtd, cc), lambda i, b: (0, b)),
            pl.BlockSpec((8, gtd), lambda i, b: (0, 0)),
            pl.BlockSpec((MB, 16), lambda i, b: (i, 0)),
        ],
        out_specs=[
            pl.BlockSpec((MB, cc), lambda i, b: (i, b)),
            pl.BlockSpec((1, MB, cc), lambda i, b: (b, i, 0)),
        ],
        out_shape=[
            jax.ShapeDtypeStruct((m, c), F32),
            jax.ShapeDtypeStruct((cb, m, cc), F32),
        ],
    )(a, w_rest, w_pool, pooled, dis)


def _tc_post(seg, h, dis, bias, cb):
    m, c = h.shape
    cc = c // cb

    def body(s_ref, h_ref, d_ref, b_ref, o_ref):
        s = s_ref[0, 0] + s_ref[1, 0]
        d = d_ref[:, :1]
        o_ref[...] = jax.nn.relu(d * s + d * d * h_ref[...] + b_ref[...])

    grid = (pl.cdiv(m, MB), cb)
    return pl.pallas_call(
        body,
        grid=grid,
        in_specs=[
            pl.BlockSpec((NC, 1, MB, cc), lambda i, b: (0, b, i, 0)),
            pl.BlockSpec((MB, cc), lambda i, b: (i, b)),
            pl.BlockSpec((MB, 16), lambda i, b: (i, 0)),
            pl.BlockSpec((1, cc), lambda i, b: (0, b)),
        ],
        out_specs=pl.BlockSpec((MB, cc), lambda i, b: (i, b)),
        out_shape=jax.ShapeDtypeStruct((m, c), F32),
    )(seg, h, dis, bias)


def _tc_knn3(pos_y, pos_x):
    ny = pos_y.shape[0]
    nx = pos_x.shape[0]

    def body(py_ref, px_ref, i_ref, w_ref):
        py = py_ref[...]
        px = px_ref[...]
        py2 = jnp.sum(py * py, axis=1, keepdims=True)
        px2 = lax.dot_general(jnp.ones((1, 3), F32), px * px,
                              (((1,), (1,)), ((), ())),
                              preferred_element_type=F32, precision=HIGH)
        cross = lax.dot_general(py, px, (((1,), (1,)), ((), ())),
                                preferred_element_type=F32, precision=HIGH)
        d = py2 - 2.0 * cross + px2
        col = lax.broadcasted_iota(jnp.int32, (MB, nx), 1)
        idxs = []
        ws = []
        for _ in range(3):
            mv = jnp.min(d, axis=1, keepdims=True)
            am = jnp.min(jnp.where(d == mv, col, nx), axis=1, keepdims=True)
            idxs.append(am)
            ws.append(1.0 / jnp.maximum(mv, 1e-16))
            d = jnp.where(col == am, 1e30, d)
        i_ref[...] = jnp.concatenate(idxs, axis=1)
        wst = jnp.concatenate(ws, axis=1)
        w_ref[...] = wst / jnp.sum(wst, axis=1, keepdims=True)

    grid = (pl.cdiv(ny, MB),)
    return pl.pallas_call(
        body,
        grid=grid,
        in_specs=[
            pl.BlockSpec((MB, 3), lambda i: (i, 0)),
            pl.BlockSpec((nx, 3), lambda i: (0, 0)),
        ],
        out_specs=[
            pl.BlockSpec((MB, 3), lambda i: (i, 0)),
            pl.BlockSpec((MB, 3), lambda i: (i, 0)),
        ],
        out_shape=[
            jax.ShapeDtypeStruct((ny, 3), jnp.int32),
            jax.ShapeDtypeStruct((ny, 3), F32),
        ],
    )(pos_y, pos_x)


def _tc_wsum(feats, wn):
    _, ny, cc = feats.shape

    def body(f_ref, w_ref, o_ref):
        o_ref[...] = (f_ref[0] * w_ref[:, 0:1] + f_ref[1] * w_ref[:, 1:2]
                      + f_ref[2] * w_ref[:, 2:3])

    grid = (pl.cdiv(ny, MB),)
    return pl.pallas_call(
        body,
        grid=grid,
        in_specs=[
            pl.BlockSpec((3, MB, cc), lambda i: (0, i, 0)),
            pl.BlockSpec((MB, 3), lambda i: (i, 0)),
        ],
        out_specs=pl.BlockSpec((MB, cc), lambda i: (i, 0)),
        out_shape=jax.ShapeDtypeStruct((ny, cc), F32),
    )(feats, wn)


def _tc_mlp_gp(x2pos, lins, bns):
    (w1, b1), (w2, b2), (w3, b3) = lins
    (g1, e1), (g2, e2) = bns
    gtd = w3.shape[1]

    def body(x_ref, w1_ref, b1_ref, g1_ref, e1_ref, w2_ref, b2_ref, g2_ref,
             e2_ref, w3_ref, b3_ref, o_ref):
        def bn_relu(h, g_ref, e_ref):
            mu = jnp.mean(h, axis=0, keepdims=True)
            var = jnp.mean((h - mu) ** 2, axis=0, keepdims=True)
            h = (h - mu) / jnp.sqrt(var + 1e-5) * g_ref[...] + e_ref[...]
            return jax.nn.relu(h)

        h = jnp.dot(x_ref[...], w1_ref[...], preferred_element_type=F32,
                    precision=HIGH) + b1_ref[...]
        h = bn_relu(h, g1_ref, e1_ref)
        h = jnp.dot(h, w2_ref[...], preferred_element_type=F32,
                    precision=HIGH) + b2_ref[...]
        h = bn_relu(h, g2_ref, e2_ref)
        h = jnp.dot(h, w3_ref[...], preferred_element_type=F32,
                    precision=HIGH) + b3_ref[...]
        pooled = jnp.max(h, axis=0, keepdims=True)
        o_ref[...] = jnp.broadcast_to(pooled, (8, gtd))

    args = (x2pos, w1, b1.reshape(1, -1), g1.reshape(1, -1), e1.reshape(1, -1),
            w2, b2.reshape(1, -1), g2.reshape(1, -1), e2.reshape(1, -1),
            w3, b3.reshape(1, -1))
    return pl.pallas_call(
        body,
        out_shape=jax.ShapeDtypeStruct((8, gtd), F32),
    )(*args)


def _tc_head(h, w1, b1, w2, b2):
    m = h.shape[0]
    co = w2.shape[1]

    def body(h_ref, w1_ref, b1_ref, w2_ref, b2_ref, o_ref):
        t = jax.nn.relu(jnp.dot(h_ref[...], w1_ref[...],
                                preferred_element_type=F32, precision=HIGH)
                        + b1_ref[...])
        o_ref[...] = jnp.dot(t, w2_ref[...], preferred_element_type=F32,
                             precision=HIGH) + b2_ref[...]

    grid = (pl.cdiv(m, MB),)
    return pl.pallas_call(
        body,
        grid=grid,
        in_specs=[
            pl.BlockSpec((MB, w1.shape[0]), lambda i: (i, 0)),
            pl.BlockSpec(w1.shape, lambda i: (0, 0)),
            pl.BlockSpec((1, w1.shape[1]), lambda i: (0, 0)),
            pl.BlockSpec(w2.shape, lambda i: (0, 0)),
            pl.BlockSpec((1, co), lambda i: (0, 0)),
        ],
        out_specs=pl.BlockSpec((MB, co), lambda i: (i, 0)),
        out_shape=jax.ShapeDtypeStruct((m, co), F32),
    )(h, w1, b1.reshape(1, -1), w2, b2.reshape(1, -1))


def _pad_edges(ei, n):
    e = ei.shape[1]
    ep = _rup(e, NW * EK)
    src = jnp.concatenate([ei[0].astype(jnp.int32),
                           jnp.zeros((ep - e,), jnp.int32)])
    dst = jnp.concatenate([ei[1].astype(jnp.int32),
                           jnp.full((ep - e,), n, jnp.int32)])
    return src, dst


def _pad_idx(idx):
    mpad = _rup(idx.shape[0], NW * EK)
    return jnp.concatenate([idx.astype(jnp.int32),
                            jnp.zeros((mpad - idx.shape[0],), jnp.int32)])


def _pad_cols(a, cc):
    if a.shape[1] == cc:
        return a
    return jnp.concatenate(
        [a, jnp.zeros((a.shape[0], cc - a.shape[1]), F32)], axis=1)


def _conv_block(a, convs, src, dst, dis, n, cbs):
    h_act = a
    for (w, b), cb in zip(convs, cbs):
        c = w.shape[1]
        cc = c // cb
        h, hp = _tc_mm(h_act, w, dis, cb)
        if cb % 2 == 0 and _npad(n) * cc <= 900000:
            seg = _sc_conv_scatter_split(hp.reshape(cb * n, cc), src, dst,
                                         n, cb, cc)
            h_act = _tc_post_split(seg, h, dis, b.reshape(1, -1), cb)
        elif cb == 1 and _npad(n) * cc <= 700000:
            seg = _sc_conv_scatter4(hp.reshape(n, cc), src, dst, n, cc)
            h_act = _tc_post(seg, h, dis, b.reshape(1, -1), cb)
        else:
            seg = _sc_conv_scatter(hp.reshape(cb * n, cc), src, dst, n, cb, cc)
            h_act = _tc_post(seg, h, dis, b.reshape(1, -1), cb)
    return h_act


def kernel(x, pos, params, batch, idx0, idx1, edge_index0, edge_index1,
           edge_index2):
    n0 = x.shape[0]
    n1 = idx0.shape[0]
    n2 = idx1.shape[0]
    in_c = x.shape[1]

    src0, dst0 = _pad_edges(edge_index0, n0)
    src1, dst1 = _pad_edges(edge_index1, n1)
    src2, dst2 = _pad_edges(edge_index2, n2)

    dis0 = _tc_dis(_sc_degree(dst0, n0), n0)
    dis1 = _tc_dis(_sc_degree(dst1, n1), n1)
    dis2 = _tc_dis(_sc_degree(dst2, n2), n2)

    h = _conv_block(x, params['sa1'], src0, dst0, dis0, n0, cbs=[1, 1, 1])

    tbl = _pad_cols(jnp.concatenate([h, pos], axis=1), 80)
    g1 = _sc_gather(tbl, _pad_idx(idx0), 80)[:n1]
    x1 = g1[:, :h.shape[1]]
    pos1 = g1[:, h.shape[1]:h.shape[1] + 3]
    a1 = g1[:, :h.shape[1] + 3]

    h = _conv_block(a1, params['sa2'], src1, dst1, dis1, n1, cbs=[1, 1, 1])

    tbl = _pad_cols(jnp.concatenate([h, pos1], axis=1), 144)
    g2 = _sc_gather(tbl, _pad_idx(idx1), 144)[:n2]
    x2pos = g2[:, :h.shape[1] + 3]
    pos2 = g2[:, h.shape[1]:h.shape[1] + 3]

    xb = _conv_block(x2pos, params['bn'], src2, dst2, dis2, n2, cbs=[1, 1, 1])

    pooled = _tc_mlp_gp(x2pos, params['gp_lin'], params['gp_bn'])

    idxk, wn = _tc_knn3(pos1, pos2)
    flat = _pad_idx(jnp.transpose(idxk).reshape(-1))
    feats = _sc_gather_wide(xb, flat, xb.shape[1])[:3 * n1].reshape(3, n1, -1)
    interp_xb = _tc_wsum(feats, wn)
    a_small = jnp.concatenate([interp_xb, x1], axis=1)

    (w, b) = params['fp2'][0]
    gtd = pooled.shape[1]
    h, hp = _tc_mm_fp2(a_small, w[gtd:], w[:gtd], pooled, dis1, cb=4)
    cc = w.shape[1] // 4
    seg = _sc_conv_scatter_split(hp.reshape(4 * n1, cc), src1, dst1, n1, 4, cc)
    h = _tc_post_split(seg, h, dis1, b.reshape(1, -1), 4)
    h = _conv_block(h, params['fp2'][1:], src1, dst1, dis1, n1, cbs=[4, 2])

    idxk, wn = _tc_knn3(pos, pos1)
    flat = _pad_idx(jnp.transpose(idxk).reshape(-1))
    feats = _sc_gather_wide(h, flat, h.shape[1])[:3 * n0].reshape(3, n0, -1)
    interp = _tc_wsum(feats, wn)
    a0 = jnp.concatenate([interp, x[:, :in_c]], axis=1)

    h = _conv_block(a0, params['fp1'], src0, dst0, dis0, n0, cbs=[2, 2, 1])

    (w1, b1), (w2, b2) = params['head']
    return _tc_head(h, w1, b1, w2, b2)
